# baseline (speedup 1.0000x reference)
import functools

import jax
import jax.numpy as jnp
from jax import lax
from jax.experimental import pallas as pl
from jax.experimental.pallas import tpu as pltpu, tpu_sc as plsc

H = 64
HH = 32
TRASH = 64
_PREC = lax.Precision.HIGHEST


def _dot(a, b):
    return jnp.dot(a, b, preferred_element_type=jnp.float32, precision=_PREC)


def _proj_body(x_ref, w_ref, b_ref, o_ref):
    y = _dot(x_ref[...], w_ref[...]) + b_ref[...]
    o_ref[0] = y[:, :HH]
    o_ref[1] = y[:, HH:]


def _proj(x, wT, b, R=1000):
    n, d = x.shape
    return pl.pallas_call(
        _proj_body,
        grid=(n // R,),
        in_specs=[pl.BlockSpec((R, d), lambda i: (i, 0)),
                  pl.BlockSpec((d, H), lambda i: (0, 0)),
                  pl.BlockSpec((1, H), lambda i: (0, 0))],
        out_specs=pl.BlockSpec((2, R, HH), lambda i: (0, i, 0)),
        out_shape=jax.ShapeDtypeStruct((2, n, HH), jnp.float32),
    )(x, wT, b.reshape(1, H))


def _cat(ref):
    return jnp.concatenate([ref[0], ref[1]], axis=1)


def _mean(s_ref, cnt_ref):
    return _cat(s_ref) * (1.0 / jnp.maximum(cnt_ref[...], 1.0))


def _comb1_body(relu, s_ref, cnt_ref, x_ref, a_ref, bmat_ref, bias_ref, o_ref):
    y = _dot(_mean(s_ref, cnt_ref), a_ref[...]) + _dot(_cat(x_ref), bmat_ref[...]) + bias_ref[...]
    if relu:
        y = jnp.maximum(y, 0.0)
    o_ref[0] = y[:, :HH]
    o_ref[1] = y[:, HH:]


def _comb1(s, cnt, x, aT, bT, bias, relu, R=1000):
    n = x.shape[1]
    return pl.pallas_call(
        functools.partial(_comb1_body, relu),
        grid=(n // R,),
        in_specs=[pl.BlockSpec((2, R, HH), lambda i: (0, i, 0)),
                  pl.BlockSpec((R, 1), lambda i: (i, 0)),
                  pl.BlockSpec((2, R, HH), lambda i: (0, i, 0)),
                  pl.BlockSpec((H, H), lambda i: (0, 0)),
                  pl.BlockSpec((H, H), lambda i: (0, 0)),
                  pl.BlockSpec((1, H), lambda i: (0, 0))],
        out_specs=pl.BlockSpec((2, R, HH), lambda i: (0, i, 0)),
        out_shape=jax.ShapeDtypeStruct((2, n, HH), jnp.float32),
    )(s, cnt, x, aT, bT, bias.reshape(1, H))


def _comb2_body(relu, s1_ref, c1_ref, s2_ref, c2_ref, x_ref, a1_ref, a2_ref,
                bmat_ref, bias_ref, o_ref):
    y = (_dot(_mean(s1_ref, c1_ref), a1_ref[...])
         + _dot(_mean(s2_ref, c2_ref), a2_ref[...])
         + _dot(_cat(x_ref), bmat_ref[...]) + bias_ref[...])
    if relu:
        y = jnp.maximum(y, 0.0)
    o_ref[0] = y[:, :HH]
    o_ref[1] = y[:, HH:]


def _comb2(s1, c1, s2, c2, x, a1T, a2T, bT, bias, relu, R=1000):
    n = x.shape[1]
    return pl.pallas_call(
        functools.partial(_comb2_body, relu),
        grid=(n // R,),
        in_specs=[pl.BlockSpec((2, R, HH), lambda i: (0, i, 0)),
                  pl.BlockSpec((R, 1), lambda i: (i, 0)),
                  pl.BlockSpec((2, R, HH), lambda i: (0, i, 0)),
                  pl.BlockSpec((R, 1), lambda i: (i, 0)),
                  pl.BlockSpec((2, R, HH), lambda i: (0, i, 0)),
                  pl.BlockSpec((H, H), lambda i: (0, 0)),
                  pl.BlockSpec((H, H), lambda i: (0, 0)),
                  pl.BlockSpec((H, H), lambda i: (0, 0)),
                  pl.BlockSpec((1, H), lambda i: (0, 0))],
        out_specs=pl.BlockSpec((2, R, HH), lambda i: (0, i, 0)),
        out_shape=jax.ShapeDtypeStruct((2, n, HH), jnp.float32),
    )(s1, c1, s2, c2, x, a1T, a2T, bT, bias.reshape(1, H))


def _colmean_body(inv_n, s_ref, cnt_ref, x_ref, o1_ref, o2_ref):
    @pl.when(pl.program_id(0) == 0)
    def _():
        o1_ref[...] = jnp.zeros_like(o1_ref)
        o2_ref[...] = jnp.zeros_like(o2_ref)

    o1_ref[...] += jnp.sum(_mean(s_ref, cnt_ref), axis=0, keepdims=True) * inv_n
    o2_ref[...] += jnp.sum(_cat(x_ref), axis=0, keepdims=True) * inv_n


def _colmean(s, cnt, x, R=1000):
    n = x.shape[1]
    return pl.pallas_call(
        functools.partial(_colmean_body, 1.0 / n),
        grid=(n // R,),
        in_specs=[pl.BlockSpec((2, R, HH), lambda i: (0, i, 0)),
                  pl.BlockSpec((R, 1), lambda i: (i, 0)),
                  pl.BlockSpec((2, R, HH), lambda i: (0, i, 0))],
        out_specs=[pl.BlockSpec((1, H), lambda i: (0, 0)),
                   pl.BlockSpec((1, H), lambda i: (0, 0))],
        out_shape=[jax.ShapeDtypeStruct((1, H), jnp.float32),
                   jax.ShapeDtypeStruct((1, H), jnp.float32)],
    )(s, cnt, x)


def _final_body(s1_ref, c1_ref, s2_ref, c2_ref, x_ref, amat_ref, const_ref, o_ref):
    y = (_dot(_mean(s1_ref, c1_ref), amat_ref[:, 0:1])
         + _dot(_mean(s2_ref, c2_ref), amat_ref[:, 1:2])
         + _dot(_cat(x_ref), amat_ref[:, 2:3]) + const_ref[...])
    o_ref[...] = y


def _final(s1, c1, s2, c2, x, amat, const, R=1000):
    n = x.shape[1]
    return pl.pallas_call(
        _final_body,
        grid=(n // R,),
        in_specs=[pl.BlockSpec((2, R, HH), lambda i: (0, i, 0)),
                  pl.BlockSpec((R, 1), lambda i: (i, 0)),
                  pl.BlockSpec((2, R, HH), lambda i: (0, i, 0)),
                  pl.BlockSpec((R, 1), lambda i: (i, 0)),
                  pl.BlockSpec((2, R, HH), lambda i: (0, i, 0)),
                  pl.BlockSpec((H, 3), lambda i: (0, 0)),
                  pl.BlockSpec((1, 1), lambda i: (0, 0))],
        out_specs=pl.BlockSpec((R, 1), lambda i: (i, 0)),
        out_shape=jax.ShapeDtypeStruct((n, 1), jnp.float32),
    )(s1, c1, s2, c2, x, amat, const)


def _agg(x_s, e, n_dst):
    x = jnp.concatenate([x_s[0], x_s[1]], axis=1)
    msg = jnp.take(x, e[0], axis=0)
    s = jax.ops.segment_sum(msg, e[1], num_segments=n_dst + TRASH)
    return jnp.stack([s[:, :HH], s[:, HH:]])


def _counts(e, n_dst):
    ones = jnp.ones((e.shape[1],), dtype=jnp.float32)
    return jax.ops.segment_sum(ones, e[1], num_segments=n_dst + TRASH).reshape(-1, 1)


def _pad_edges(e, n_src, n_dst, e_pad):
    pad = e_pad - e.shape[1]
    idx = jnp.arange(pad, dtype=jnp.int32)
    return jnp.concatenate(
        [e, jnp.stack([idx % n_src, n_dst + (idx % TRASH)])], axis=1)


def kernel(x_patient, x_med, x_com, edge_pm, edge_mp, edge_pc, edge_cp,
           proj_Wp, proj_bp, proj_Wm, proj_bm, proj_Wc, proj_bc,
           Wl, bl, Wr, cls_W, cls_b):
    n_p, n_m, n_c = x_patient.shape[0], x_med.shape[0], x_com.shape[0]
    EPM_PAD, EPC_PAD = 802816, 200704

    e_pm = _pad_edges(edge_pm, n_p, n_m, EPM_PAD)
    e_mp = _pad_edges(edge_mp, n_m, n_p, EPM_PAD)
    e_pc = _pad_edges(edge_pc, n_p, n_c, EPC_PAD)
    e_cp = _pad_edges(edge_cp, n_c, n_p, EPC_PAD)

    xp = _proj(x_patient, proj_Wp.T, proj_bp)
    xm = _proj(x_med, proj_Wm.T, proj_bm)
    xc = _proj(x_com, proj_Wc.T, proj_bc)

    cnt_m = _counts(e_pm, n_m)
    cnt_p1 = _counts(e_mp, n_p)
    cnt_c = _counts(e_pc, n_c)
    cnt_p2 = _counts(e_cp, n_p)

    s_pm = _agg(xp, e_pm, n_m)
    s_mp = _agg(xm, e_mp, n_p)
    s_pc = _agg(xp, e_pc, n_c)
    s_cp = _agg(xc, e_cp, n_p)
    xm1 = _comb1(s_pm, cnt_m, xm, Wl[0, 0].T, Wr[0, 0].T, bl[0, 0], relu=True)
    xc1 = _comb1(s_pc, cnt_c, xc, Wl[0, 2].T, Wr[0, 2].T, bl[0, 2], relu=True)
    xp1 = _comb2(s_mp, cnt_p1, s_cp, cnt_p2, xp, Wl[0, 1].T, Wl[0, 3].T,
                 (Wr[0, 1] + Wr[0, 3]).T, bl[0, 1] + bl[0, 3], relu=True)

    s_pm2 = _agg(xp1, e_pm, n_m)
    s_mp2 = _agg(xm1, e_mp, n_p)
    s_pc2 = _agg(xp1, e_pc, n_c)
    s_cp2 = _agg(xc1, e_cp, n_p)

    cm_mean_m, cm_x_m = _colmean(s_pm2, cnt_m, xm1)
    cm_mean_c, cm_x_c = _colmean(s_pc2, cnt_c, xc1)

    w1, w2, w3 = cls_W[0, :H], cls_W[0, H:2 * H], cls_W[0, 2 * H:]
    a1 = Wl[1, 1].T @ w1
    a2 = Wl[1, 3].T @ w1
    av = (Wr[1, 1] + Wr[1, 3]).T @ w1
    g1, g2 = Wl[1, 0].T @ w2, Wr[1, 0].T @ w2
    h1, h2 = Wl[1, 2].T @ w3, Wr[1, 2].T @ w3
    s_scalar = (cm_mean_m[0] @ g1 + cm_x_m[0] @ g2
                + cm_mean_c[0] @ h1 + cm_x_c[0] @ h2
                + (bl[1, 1] + bl[1, 3]) @ w1 + bl[1, 0] @ w2 + bl[1, 2] @ w3
                + cls_b[0])
    amat = jnp.stack([a1, a2, av], axis=1)

    return _final(s_mp2, cnt_p1, s_cp2, cnt_p2, xp1, amat,
                  s_scalar.reshape(1, 1))


# speedup vs baseline: 7.3209x; 7.3209x over previous
import functools

import jax
import jax.numpy as jnp
from jax import lax
from jax.experimental import pallas as pl
from jax.experimental.pallas import tpu as pltpu, tpu_sc as plsc

H = 64
HH = 32
_PREC = lax.Precision.HIGHEST
_NT = 16
_ROUNDS = 49


def _trash(n):
    t = (-n) % 128
    return t if t else 128


def _dot(a, b):
    return jnp.dot(a, b, preferred_element_type=jnp.float32, precision=_PREC)


def _proj_body(x_ref, w_ref, b_ref, o_ref):
    y = _dot(x_ref[...], w_ref[...]) + b_ref[...]
    o_ref[0] = y[:, :HH]
    o_ref[1] = y[:, HH:]


def _proj(x, wT, b, R=1000):
    n, d = x.shape
    return pl.pallas_call(
        _proj_body,
        grid=(n // R,),
        in_specs=[pl.BlockSpec((R, d), lambda i: (i, 0)),
                  pl.BlockSpec((d, H), lambda i: (0, 0)),
                  pl.BlockSpec((1, H), lambda i: (0, 0))],
        out_specs=pl.BlockSpec((2, R, HH), lambda i: (0, i, 0)),
        out_shape=jax.ShapeDtypeStruct((2, n + _trash(n), HH), jnp.float32),
    )(x, wT, b.reshape(1, H))


def _cat(ref):
    return jnp.concatenate([ref[0], ref[1]], axis=1)


def _mean(s_ref, cnt_ref):
    return _cat(s_ref) * (1.0 / jnp.maximum(cnt_ref[...], 1.0))


def _comb1_body(relu, s_ref, cnt_ref, x_ref, a_ref, bmat_ref, bias_ref, o_ref):
    y = _dot(_mean(s_ref, cnt_ref), a_ref[...]) + _dot(_cat(x_ref), bmat_ref[...]) + bias_ref[...]
    if relu:
        y = jnp.maximum(y, 0.0)
    o_ref[0] = y[:, :HH]
    o_ref[1] = y[:, HH:]


def _comb1(s, cnt, x, aT, bT, bias, relu, n, R=1000):
    return pl.pallas_call(
        functools.partial(_comb1_body, relu),
        grid=(n // R,),
        in_specs=[pl.BlockSpec((2, R, HH), lambda i: (0, i, 0)),
                  pl.BlockSpec((R, 1), lambda i: (i, 0)),
                  pl.BlockSpec((2, R, HH), lambda i: (0, i, 0)),
                  pl.BlockSpec((H, H), lambda i: (0, 0)),
                  pl.BlockSpec((H, H), lambda i: (0, 0)),
                  pl.BlockSpec((1, H), lambda i: (0, 0))],
        out_specs=pl.BlockSpec((2, R, HH), lambda i: (0, i, 0)),
        out_shape=jax.ShapeDtypeStruct((2, n + _trash(n), HH), jnp.float32),
    )(s, cnt, x, aT, bT, bias.reshape(1, H))


def _comb2_body(relu, s1_ref, c1_ref, s2_ref, c2_ref, x_ref, a1_ref, a2_ref,
                bmat_ref, bias_ref, o_ref):
    y = (_dot(_mean(s1_ref, c1_ref), a1_ref[...])
         + _dot(_mean(s2_ref, c2_ref), a2_ref[...])
         + _dot(_cat(x_ref), bmat_ref[...]) + bias_ref[...])
    if relu:
        y = jnp.maximum(y, 0.0)
    o_ref[0] = y[:, :HH]
    o_ref[1] = y[:, HH:]


def _comb2(s1, c1, s2, c2, x, a1T, a2T, bT, bias, relu, n, R=1000):
    return pl.pallas_call(
        functools.partial(_comb2_body, relu),
        grid=(n // R,),
        in_specs=[pl.BlockSpec((2, R, HH), lambda i: (0, i, 0)),
                  pl.BlockSpec((R, 1), lambda i: (i, 0)),
                  pl.BlockSpec((2, R, HH), lambda i: (0, i, 0)),
                  pl.BlockSpec((R, 1), lambda i: (i, 0)),
                  pl.BlockSpec((2, R, HH), lambda i: (0, i, 0)),
                  pl.BlockSpec((H, H), lambda i: (0, 0)),
                  pl.BlockSpec((H, H), lambda i: (0, 0)),
                  pl.BlockSpec((H, H), lambda i: (0, 0)),
                  pl.BlockSpec((1, H), lambda i: (0, 0))],
        out_specs=pl.BlockSpec((2, R, HH), lambda i: (0, i, 0)),
        out_shape=jax.ShapeDtypeStruct((2, n + _trash(n), HH), jnp.float32),
    )(s1, c1, s2, c2, x, a1T, a2T, bT, bias.reshape(1, H))


def _colmean_body(inv_n, s_ref, cnt_ref, x_ref, o1_ref, o2_ref):
    @pl.when(pl.program_id(0) == 0)
    def _():
        o1_ref[...] = jnp.zeros_like(o1_ref)
        o2_ref[...] = jnp.zeros_like(o2_ref)

    o1_ref[...] += jnp.sum(_mean(s_ref, cnt_ref), axis=0, keepdims=True) * inv_n
    o2_ref[...] += jnp.sum(_cat(x_ref), axis=0, keepdims=True) * inv_n


def _colmean(s, cnt, x, n, R=1000):
    return pl.pallas_call(
        functools.partial(_colmean_body, 1.0 / n),
        grid=(n // R,),
        in_specs=[pl.BlockSpec((2, R, HH), lambda i: (0, i, 0)),
                  pl.BlockSpec((R, 1), lambda i: (i, 0)),
                  pl.BlockSpec((2, R, HH), lambda i: (0, i, 0))],
        out_specs=[pl.BlockSpec((1, H), lambda i: (0, 0)),
                   pl.BlockSpec((1, H), lambda i: (0, 0))],
        out_shape=[jax.ShapeDtypeStruct((1, H), jnp.float32),
                   jax.ShapeDtypeStruct((1, H), jnp.float32)],
    )(s, cnt, x)


def _final_body(s1_ref, c1_ref, s2_ref, c2_ref, x_ref, amat_ref, const_ref, o_ref):
    y = (_dot(_mean(s1_ref, c1_ref), amat_ref[:, 0:1])
         + _dot(_mean(s2_ref, c2_ref), amat_ref[:, 1:2])
         + _dot(_cat(x_ref), amat_ref[:, 2:3]) + const_ref[...])
    o_ref[...] = y


def _final(s1, c1, s2, c2, x, amat, const, n, R=1000):
    return pl.pallas_call(
        _final_body,
        grid=(n // R,),
        in_specs=[pl.BlockSpec((2, R, HH), lambda i: (0, i, 0)),
                  pl.BlockSpec((R, 1), lambda i: (i, 0)),
                  pl.BlockSpec((2, R, HH), lambda i: (0, i, 0)),
                  pl.BlockSpec((R, 1), lambda i: (i, 0)),
                  pl.BlockSpec((2, R, HH), lambda i: (0, i, 0)),
                  pl.BlockSpec((H, 3), lambda i: (0, 0)),
                  pl.BlockSpec((1, 1), lambda i: (0, 0))],
        out_specs=pl.BlockSpec((R, 1), lambda i: (i, 0)),
        out_shape=jax.ShapeDtypeStruct((n, 1), jnp.float32),
    )(s1, c1, s2, c2, x, amat, const)


_KR = 2


def _sc_agg_body(n_p, n_m, n_c, xp_s, xm_s, xc_s, spm, dpm, smp, dmp, spc,
                 dpc, scp, dcp, zeros2d, o_pm, o_mp, o_pc, o_cp,
                 acc, gbuf, dbuf, rows, semg, sems):
    c = lax.axis_index("c")
    t = lax.axis_index("s")
    np_f, nm_f, nc_f = n_p + _trash(n_p), n_m + _trash(n_m), n_c + _trash(n_c)

    def zero(n):
        k = n // _NT
        pltpu.sync_copy(zeros2d.at[pl.ds(t * k, k)], acc.at[pl.ds(t * k, k)])

    def writeback(n, out):
        k = n // _NT
        pltpu.sync_copy(acc.at[pl.ds(t * k, k)], out.at[c, pl.ds(t * k, k)])

    def edge_loop(s2d, d2d, tbl):
        rounds = s2d.shape[0] // (_NT * _KR)
        base = t * rounds * _KR

        def body(g, carry):
            r0 = base + g * _KR
            pltpu.sync_copy(s2d.at[pl.ds(r0, _KR)], gbuf)
            pltpu.sync_copy(d2d.at[pl.ds(r0, _KR)], dbuf)
            gds = [pltpu.async_copy(tbl.at[gbuf.at[j]],
                                    rows.at[pl.ds(j * 128, 128)], semg)
                   for j in range(_KR)]
            for dd in gds:
                dd.wait()
            sds = [pltpu.async_copy(rows.at[pl.ds(j * 128, 128)],
                                    acc.at[dbuf.at[j]], sems, add=True)
                   for j in range(_KR)]
            for dd in sds:
                dd.wait()
            return carry

        lax.fori_loop(0, rounds, body, 0)

    bar = plsc.subcore_barrier
    for src2d, dst2d, x_hbm, n_out, out in (
            (spm, dpm, xp_s, nm_f, o_pm),
            (spc, dpc, xp_s, nc_f, o_pc),
            (smp, dmp, xm_s, np_f, o_mp),
            (scp, dcp, xc_s, np_f, o_cp)):
        zero(n_out)
        bar()
        edge_loop(src2d, dst2d, x_hbm.at[c])
        bar()
        writeback(n_out, out)
        bar()


def _sc_agg(xp_s, xm_s, xc_s, eidx, zeros2d, n_p, n_m, n_c):
    mesh = plsc.VectorSubcoreMesh(core_axis_name="c", subcore_axis_name="s")
    np_f, nm_f, nc_f = n_p + _trash(n_p), n_m + _trash(n_m), n_c + _trash(n_c)
    f = pl.kernel(
        functools.partial(_sc_agg_body, n_p, n_m, n_c),
        compiler_params=pltpu.CompilerParams(use_tc_tiling_on_sc=False),
        out_type=[jax.ShapeDtypeStruct((2, nm_f, HH), jnp.float32),
                  jax.ShapeDtypeStruct((2, np_f, HH), jnp.float32),
                  jax.ShapeDtypeStruct((2, nc_f, HH), jnp.float32),
                  jax.ShapeDtypeStruct((2, np_f, HH), jnp.float32)],
        mesh=mesh,
        scratch_types=[
            pltpu.VMEM_SHARED((np_f, HH), jnp.float32),
            pltpu.VMEM((_KR, 128), jnp.int32),
            pltpu.VMEM((_KR, 128), jnp.int32),
            pltpu.VMEM((_KR * 128, HH), jnp.float32),
            pltpu.SemaphoreType.DMA,
            pltpu.SemaphoreType.DMA,
        ])
    return f(xp_s, xm_s, xc_s, *eidx, zeros2d)


def _sc_counts_body(n_p, n_m, n_c, dpm, dmp, dpc, dcp, zeros1,
                    o_m, o_p1, o_c, o_p2, acc_a, acc_b, dbuf, ones, sem, b1d):
    c = lax.axis_index("c")
    t = lax.axis_index("s")
    np_f, nm_f, nc_f = n_p + _trash(n_p), n_m + _trash(n_m), n_c + _trash(n_c)

    for i in range(8):
        ones[pl.ds(i * 16, 16)] = jnp.ones((16,), jnp.float32)

    def zero1(acc, n):
        k = n // _NT
        pltpu.sync_copy(zeros1.at[pl.ds(t * k, k)], b1d.at[pl.ds(0, k)])
        pltpu.sync_copy(b1d.at[pl.ds(0, k)], acc.at[pl.ds(t * k, k)])

    def wb1(acc, n, out):
        k = n // _NT
        pltpu.sync_copy(acc.at[pl.ds(t * k, k)], b1d.at[pl.ds(0, k)])
        pltpu.sync_copy(b1d.at[pl.ds(0, k)], out.at[pl.ds(t * k, k)])

    def cnt_loop(d2d, kr, acc):
        base = t * _ROUNDS * kr

        def body(g, carry):
            r0 = base + g * kr
            pltpu.sync_copy(d2d.at[pl.ds(r0, kr)], dbuf.at[pl.ds(0, kr)])
            ds_ = [pltpu.async_copy(ones, acc.at[dbuf.at[j]], sem, add=True)
                   for j in range(kr)]
            for dd in ds_:
                dd.wait()
            return carry

        lax.fori_loop(0, _ROUNDS, body, 0)

    bar = plsc.subcore_barrier
    zero1(acc_a, np_f)
    zero1(acc_b, nm_f)
    bar()

    @pl.when(c == 0)
    def _():
        cnt_loop(dpm, 8, acc_b)
        cnt_loop(dmp, 8, acc_a)

    @pl.when(c == 1)
    def _():
        cnt_loop(dpc, 2, acc_b)
        cnt_loop(dcp, 2, acc_a)

    bar()

    @pl.when(c == 0)
    def _():
        wb1(acc_b, nm_f, o_m)
        wb1(acc_a, np_f, o_p1)

    @pl.when(c == 1)
    def _():
        wb1(acc_b, nc_f, o_c)
        wb1(acc_a, np_f, o_p2)


def _sc_counts(dpm, dmp, dpc, dcp, zeros1, n_p, n_m, n_c):
    mesh = plsc.VectorSubcoreMesh(core_axis_name="c", subcore_axis_name="s")
    np_f, nm_f, nc_f = n_p + _trash(n_p), n_m + _trash(n_m), n_c + _trash(n_c)
    f = pl.kernel(
        functools.partial(_sc_counts_body, n_p, n_m, n_c),
        compiler_params=pltpu.CompilerParams(use_tc_tiling_on_sc=False),
        out_type=[jax.ShapeDtypeStruct((nm_f,), jnp.float32),
                  jax.ShapeDtypeStruct((np_f,), jnp.float32),
                  jax.ShapeDtypeStruct((nc_f,), jnp.float32),
                  jax.ShapeDtypeStruct((np_f,), jnp.float32)],
        mesh=mesh,
        scratch_types=[
            pltpu.VMEM_SHARED((np_f,), jnp.float32),
            pltpu.VMEM_SHARED((nm_f,), jnp.float32),
            pltpu.VMEM((8, 128), jnp.int32),
            pltpu.VMEM((128,), jnp.float32),
            pltpu.SemaphoreType.DMA,
            pltpu.VMEM(((n_p + _trash(n_p)) // _NT,), jnp.float32),
        ])
    return f(dpm, dmp, dpc, dcp, zeros1)


def _pad_edges(e, n_src, n_dst, e_pad):
    pad = e_pad - e.shape[1]
    idx = jnp.arange(pad, dtype=jnp.int32)
    ext = jnp.concatenate(
        [e, jnp.stack([idx % n_src, n_dst + (idx % _trash(n_dst))])], axis=1)
    return ext[0].reshape(-1, 128), ext[1].reshape(-1, 128)


def kernel(x_patient, x_med, x_com, edge_pm, edge_mp, edge_pc, edge_cp,
           proj_Wp, proj_bp, proj_Wm, proj_bm, proj_Wc, proj_bc,
           Wl, bl, Wr, cls_W, cls_b):
    n_p, n_m, n_c = x_patient.shape[0], x_med.shape[0], x_com.shape[0]
    EPM_PAD = _NT * _ROUNDS * 8 * 128
    EPC_PAD = _NT * _ROUNDS * 2 * 128

    spm, dpm = _pad_edges(edge_pm, n_p, n_m, EPM_PAD)
    smp, dmp = _pad_edges(edge_mp, n_m, n_p, EPM_PAD)
    spc, dpc = _pad_edges(edge_pc, n_p, n_c, EPC_PAD)
    scp, dcp = _pad_edges(edge_cp, n_c, n_p, EPC_PAD)
    eidx = (spm, dpm, smp, dmp, spc, dpc, scp, dcp)
    zeros2d = jnp.zeros((n_p + _trash(n_p), HH), jnp.float32)
    zeros1 = jnp.zeros((n_p + _trash(n_p),), jnp.float32)

    xp = _proj(x_patient, proj_Wp.T, proj_bp)
    xm = _proj(x_med, proj_Wm.T, proj_bm)
    xc = _proj(x_com, proj_Wc.T, proj_bc)

    cnt_m, cnt_p1, cnt_c, cnt_p2 = [
        a.reshape(-1, 1)
        for a in _sc_counts(dpm, dmp, dpc, dcp, zeros1, n_p, n_m, n_c)]

    s_pm, s_mp, s_pc, s_cp = _sc_agg(xp, xm, xc, eidx, zeros2d, n_p, n_m, n_c)
    xm1 = _comb1(s_pm, cnt_m, xm, Wl[0, 0].T, Wr[0, 0].T, bl[0, 0],
                 relu=True, n=n_m)
    xc1 = _comb1(s_pc, cnt_c, xc, Wl[0, 2].T, Wr[0, 2].T, bl[0, 2],
                 relu=True, n=n_c)
    xp1 = _comb2(s_mp, cnt_p1, s_cp, cnt_p2, xp, Wl[0, 1].T, Wl[0, 3].T,
                 (Wr[0, 1] + Wr[0, 3]).T, bl[0, 1] + bl[0, 3],
                 relu=True, n=n_p)

    s_pm2, s_mp2, s_pc2, s_cp2 = _sc_agg(xp1, xm1, xc1, eidx, zeros2d,
                                         n_p, n_m, n_c)

    cm_mean_m, cm_x_m = _colmean(s_pm2, cnt_m, xm1, n=n_m)
    cm_mean_c, cm_x_c = _colmean(s_pc2, cnt_c, xc1, n=n_c)

    w1, w2, w3 = cls_W[0, :H], cls_W[0, H:2 * H], cls_W[0, 2 * H:]
    a1 = Wl[1, 1].T @ w1
    a2 = Wl[1, 3].T @ w1
    av = (Wr[1, 1] + Wr[1, 3]).T @ w1
    g1, g2 = Wl[1, 0].T @ w2, Wr[1, 0].T @ w2
    h1, h2 = Wl[1, 2].T @ w3, Wr[1, 2].T @ w3
    s_scalar = (cm_mean_m[0] @ g1 + cm_x_m[0] @ g2
                + cm_mean_c[0] @ h1 + cm_x_c[0] @ h2
                + (bl[1, 1] + bl[1, 3]) @ w1 + bl[1, 0] @ w2 + bl[1, 2] @ w3
                + cls_b[0])
    amat = jnp.stack([a1, a2, av], axis=1)

    return _final(s_mp2, cnt_p1, s_cp2, cnt_p2, xp1, amat,
                  s_scalar.reshape(1, 1), n=n_p)


# speedup vs baseline: 11.1924x; 1.5288x over previous
import functools

import jax
import jax.numpy as jnp
from jax import lax
from jax.experimental import pallas as pl
from jax.experimental.pallas import tpu as pltpu, tpu_sc as plsc

H = 64
HH = 32
_PREC = lax.Precision.HIGHEST
_NT = 16


def _trash(n):
    t = (-n) % 128
    return t if t else 128


def _dot(a, b):
    return jnp.dot(a, b, preferred_element_type=jnp.float32, precision=_PREC)


def _proj_body(x_ref, w_ref, b_ref, o_ref):
    y = _dot(x_ref[...], w_ref[...]) + b_ref[...]
    o_ref[0] = y[:, :HH]
    o_ref[1] = y[:, HH:]


def _proj(x, wT, b, R=1000):
    n, d = x.shape
    return pl.pallas_call(
        _proj_body,
        grid=(n // R,),
        in_specs=[pl.BlockSpec((R, d), lambda i: (i, 0)),
                  pl.BlockSpec((d, H), lambda i: (0, 0)),
                  pl.BlockSpec((1, H), lambda i: (0, 0))],
        out_specs=pl.BlockSpec((2, R, HH), lambda i: (0, i, 0)),
        out_shape=jax.ShapeDtypeStruct((2, n + _trash(n), HH), jnp.float32),
    )(x, wT, b.reshape(1, H))


def _cat(ref):
    return jnp.concatenate([ref[0], ref[1]], axis=1)


def _mean(s_ref, cnt_ref):
    return _cat(s_ref) * (1.0 / jnp.maximum(cnt_ref[...], 1.0))


def _comb1_body(relu, s_ref, cnt_ref, x_ref, a_ref, bmat_ref, bias_ref, o_ref):
    y = _dot(_mean(s_ref, cnt_ref), a_ref[...]) + _dot(_cat(x_ref), bmat_ref[...]) + bias_ref[...]
    if relu:
        y = jnp.maximum(y, 0.0)
    o_ref[0] = y[:, :HH]
    o_ref[1] = y[:, HH:]


def _comb1(s, cnt, x, aT, bT, bias, relu, n, R=1000):
    return pl.pallas_call(
        functools.partial(_comb1_body, relu),
        grid=(n // R,),
        in_specs=[pl.BlockSpec((2, R, HH), lambda i: (0, i, 0)),
                  pl.BlockSpec((R, 1), lambda i: (i, 0)),
                  pl.BlockSpec((2, R, HH), lambda i: (0, i, 0)),
                  pl.BlockSpec((H, H), lambda i: (0, 0)),
                  pl.BlockSpec((H, H), lambda i: (0, 0)),
                  pl.BlockSpec((1, H), lambda i: (0, 0))],
        out_specs=pl.BlockSpec((2, R, HH), lambda i: (0, i, 0)),
        out_shape=jax.ShapeDtypeStruct((2, n + _trash(n), HH), jnp.float32),
    )(s, cnt, x, aT, bT, bias.reshape(1, H))


def _comb2_body(relu, s1_ref, c1_ref, s2_ref, c2_ref, x_ref, a1_ref, a2_ref,
                bmat_ref, bias_ref, o_ref):
    y = (_dot(_mean(s1_ref, c1_ref), a1_ref[...])
         + _dot(_mean(s2_ref, c2_ref), a2_ref[...])
         + _dot(_cat(x_ref), bmat_ref[...]) + bias_ref[...])
    if relu:
        y = jnp.maximum(y, 0.0)
    o_ref[0] = y[:, :HH]
    o_ref[1] = y[:, HH:]


def _comb2(s1, c1, s2, c2, x, a1T, a2T, bT, bias, relu, n, R=1000):
    return pl.pallas_call(
        functools.partial(_comb2_body, relu),
        grid=(n // R,),
        in_specs=[pl.BlockSpec((2, R, HH), lambda i: (0, i, 0)),
                  pl.BlockSpec((R, 1), lambda i: (i, 0)),
                  pl.BlockSpec((2, R, HH), lambda i: (0, i, 0)),
                  pl.BlockSpec((R, 1), lambda i: (i, 0)),
                  pl.BlockSpec((2, R, HH), lambda i: (0, i, 0)),
                  pl.BlockSpec((H, H), lambda i: (0, 0)),
                  pl.BlockSpec((H, H), lambda i: (0, 0)),
                  pl.BlockSpec((H, H), lambda i: (0, 0)),
                  pl.BlockSpec((1, H), lambda i: (0, 0))],
        out_specs=pl.BlockSpec((2, R, HH), lambda i: (0, i, 0)),
        out_shape=jax.ShapeDtypeStruct((2, n + _trash(n), HH), jnp.float32),
    )(s1, c1, s2, c2, x, a1T, a2T, bT, bias.reshape(1, H))


def _colmean_body(inv_n, s_ref, cnt_ref, x_ref, o1_ref, o2_ref):
    @pl.when(pl.program_id(0) == 0)
    def _():
        o1_ref[...] = jnp.zeros_like(o1_ref)
        o2_ref[...] = jnp.zeros_like(o2_ref)

    o1_ref[...] += jnp.sum(_mean(s_ref, cnt_ref), axis=0, keepdims=True) * inv_n
    o2_ref[...] += jnp.sum(_cat(x_ref), axis=0, keepdims=True) * inv_n


def _colmean(s, cnt, x, n, R=1000):
    return pl.pallas_call(
        functools.partial(_colmean_body, 1.0 / n),
        grid=(n // R,),
        in_specs=[pl.BlockSpec((2, R, HH), lambda i: (0, i, 0)),
                  pl.BlockSpec((R, 1), lambda i: (i, 0)),
                  pl.BlockSpec((2, R, HH), lambda i: (0, i, 0))],
        out_specs=[pl.BlockSpec((1, H), lambda i: (0, 0)),
                   pl.BlockSpec((1, H), lambda i: (0, 0))],
        out_shape=[jax.ShapeDtypeStruct((1, H), jnp.float32),
                   jax.ShapeDtypeStruct((1, H), jnp.float32)],
    )(s, cnt, x)


def _final_body(s1_ref, c1_ref, s2_ref, c2_ref, x_ref, amat_ref, const_ref, o_ref):
    y = (_dot(_mean(s1_ref, c1_ref), amat_ref[:, 0:1])
         + _dot(_mean(s2_ref, c2_ref), amat_ref[:, 1:2])
         + _dot(_cat(x_ref), amat_ref[:, 2:3]) + const_ref[...])
    o_ref[...] = y


def _final(s1, c1, s2, c2, x, amat, const, n, R=1000):
    return pl.pallas_call(
        _final_body,
        grid=(n // R,),
        in_specs=[pl.BlockSpec((2, R, HH), lambda i: (0, i, 0)),
                  pl.BlockSpec((R, 1), lambda i: (i, 0)),
                  pl.BlockSpec((2, R, HH), lambda i: (0, i, 0)),
                  pl.BlockSpec((R, 1), lambda i: (i, 0)),
                  pl.BlockSpec((2, R, HH), lambda i: (0, i, 0)),
                  pl.BlockSpec((H, 3), lambda i: (0, 0)),
                  pl.BlockSpec((1, 1), lambda i: (0, 0))],
        out_specs=pl.BlockSpec((R, 1), lambda i: (i, 0)),
        out_shape=jax.ShapeDtypeStruct((n, 1), jnp.float32),
    )(s1, c1, s2, c2, x, amat, const)


_KR = 2


def _sc_agg_body(n_p, n_m, n_c, xp_s, xm_s, xc_s, spm, dpm, smp, dmp, spc,
                 dpc, scp, dcp, zeros2d, o_pm, o_mp, o_pc, o_cp,
                 acc, gbuf, dbuf, rows, semi, semg, sems):
    c = lax.axis_index("c")
    t = lax.axis_index("s")
    np_f, nm_f, nc_f = n_p + _trash(n_p), n_m + _trash(n_m), n_c + _trash(n_c)

    def zero(n):
        k = n // _NT
        pltpu.sync_copy(zeros2d.at[pl.ds(t * k, k)], acc.at[pl.ds(t * k, k)])

    def writeback(n, out):
        k = n // _NT
        pltpu.sync_copy(acc.at[pl.ds(t * k, k)], out.at[c, pl.ds(t * k, k)])

    def edge_loop(s2d, d2d, tbl):
        rounds = s2d.shape[0] // (_NT * _KR)
        pairs = rounds // 2
        base = t * rounds * _KR

        def fire_idx(g, slot):
            r0 = base + g * _KR
            pltpu.async_copy(s2d.at[pl.ds(r0, _KR)], gbuf.at[slot], semi)
            pltpu.async_copy(d2d.at[pl.ds(r0, _KR)], dbuf.at[slot], semi)

        def wait_idx(slot):
            pltpu.make_async_copy(s2d.at[pl.ds(base, _KR)], gbuf.at[slot], semi).wait()
            pltpu.make_async_copy(d2d.at[pl.ds(base, _KR)], dbuf.at[slot], semi).wait()

        def fire_gathers(slot):
            for j in range(_KR):
                pltpu.async_copy(tbl.at[gbuf.at[slot, j]],
                                 rows.at[slot, pl.ds(j * 128, 128)], semg)

        def wait_gathers(slot):
            for j in range(_KR):
                pltpu.make_async_copy(tbl.at[gbuf.at[slot, j]],
                                      rows.at[slot, pl.ds(j * 128, 128)], semg).wait()

        def fire_scatters(slot):
            for j in range(_KR):
                pltpu.async_copy(rows.at[slot, pl.ds(j * 128, 128)],
                                 acc.at[dbuf.at[slot, j]], sems, add=True)

        def wait_scatters(slot):
            for j in range(_KR):
                pltpu.make_async_copy(rows.at[slot, pl.ds(j * 128, 128)],
                                      acc.at[dbuf.at[slot, j]], sems).wait()

        fire_idx(0, 0)

        def body(p, carry):
            wait_idx(0)
            fire_gathers(0)
            fire_idx(2 * p + 1, 1)
            wait_gathers(0)
            fire_scatters(0)
            wait_idx(1)
            fire_gathers(1)

            @pl.when(p + 1 < pairs)
            def _():
                fire_idx(2 * p + 2, 0)

            wait_scatters(0)
            wait_gathers(1)
            fire_scatters(1)
            wait_scatters(1)
            return carry

        lax.fori_loop(0, pairs, body, 0)

    bar = plsc.subcore_barrier
    for src2d, dst2d, x_hbm, n_out, out in (
            (spm, dpm, xp_s, nm_f, o_pm),
            (spc, dpc, xp_s, nc_f, o_pc),
            (smp, dmp, xm_s, np_f, o_mp),
            (scp, dcp, xc_s, np_f, o_cp)):
        zero(n_out)
        bar()
        edge_loop(src2d, dst2d, x_hbm.at[c])
        bar()
        writeback(n_out, out)
        bar()


def _sc_agg(xp_s, xm_s, xc_s, eidx, zeros2d, n_p, n_m, n_c):
    mesh = plsc.VectorSubcoreMesh(core_axis_name="c", subcore_axis_name="s")
    np_f, nm_f, nc_f = n_p + _trash(n_p), n_m + _trash(n_m), n_c + _trash(n_c)
    f = pl.kernel(
        functools.partial(_sc_agg_body, n_p, n_m, n_c),
        compiler_params=pltpu.CompilerParams(use_tc_tiling_on_sc=False),
        out_type=[jax.ShapeDtypeStruct((2, nm_f, HH), jnp.float32),
                  jax.ShapeDtypeStruct((2, np_f, HH), jnp.float32),
                  jax.ShapeDtypeStruct((2, nc_f, HH), jnp.float32),
                  jax.ShapeDtypeStruct((2, np_f, HH), jnp.float32)],
        mesh=mesh,
        scratch_types=[
            pltpu.VMEM_SHARED((np_f, HH), jnp.float32),
            pltpu.VMEM((2, _KR, 128), jnp.int32),
            pltpu.VMEM((2, _KR, 128), jnp.int32),
            pltpu.VMEM((2, _KR * 128, HH), jnp.float32),
            pltpu.SemaphoreType.DMA,
            pltpu.SemaphoreType.DMA,
            pltpu.SemaphoreType.DMA,
        ])
    return f(xp_s, xm_s, xc_s, *eidx, zeros2d)


def _sc_counts_body(n_p, n_m, n_c, dpm, dmp, dpc, dcp, zeros1,
                    o_m, o_p1, o_c, o_p2, acc_a, acc_b, dbuf, ones, sem, b1d):
    c = lax.axis_index("c")
    t = lax.axis_index("s")
    np_f, nm_f, nc_f = n_p + _trash(n_p), n_m + _trash(n_m), n_c + _trash(n_c)

    for i in range(8):
        ones[pl.ds(i * 16, 16)] = jnp.ones((16,), jnp.float32)

    def zero1(acc, n):
        k = n // _NT
        pltpu.sync_copy(zeros1.at[pl.ds(t * k, k)], b1d.at[pl.ds(0, k)])
        pltpu.sync_copy(b1d.at[pl.ds(0, k)], acc.at[pl.ds(t * k, k)])

    def wb1(acc, n, out):
        k = n // _NT
        pltpu.sync_copy(acc.at[pl.ds(t * k, k)], b1d.at[pl.ds(0, k)])
        pltpu.sync_copy(b1d.at[pl.ds(0, k)], out.at[pl.ds(t * k, k)])

    def cnt_loop(d2d, kr, acc):
        rounds = d2d.shape[0] // (_NT * kr)
        base = t * rounds * kr

        def body(g, carry):
            r0 = base + g * kr
            pltpu.sync_copy(d2d.at[pl.ds(r0, kr)], dbuf.at[pl.ds(0, kr)])
            ds_ = [pltpu.async_copy(ones, acc.at[dbuf.at[j]], sem, add=True)
                   for j in range(kr)]
            for dd in ds_:
                dd.wait()
            return carry

        lax.fori_loop(0, rounds, body, 0)

    bar = plsc.subcore_barrier
    zero1(acc_a, np_f)
    zero1(acc_b, nm_f)
    bar()

    @pl.when(c == 0)
    def _():
        cnt_loop(dpm, 8, acc_b)
        cnt_loop(dmp, 8, acc_a)

    @pl.when(c == 1)
    def _():
        cnt_loop(dpc, 2, acc_b)
        cnt_loop(dcp, 2, acc_a)

    bar()

    @pl.when(c == 0)
    def _():
        wb1(acc_b, nm_f, o_m)
        wb1(acc_a, np_f, o_p1)

    @pl.when(c == 1)
    def _():
        wb1(acc_b, nc_f, o_c)
        wb1(acc_a, np_f, o_p2)


def _sc_counts(dpm, dmp, dpc, dcp, zeros1, n_p, n_m, n_c):
    mesh = plsc.VectorSubcoreMesh(core_axis_name="c", subcore_axis_name="s")
    np_f, nm_f, nc_f = n_p + _trash(n_p), n_m + _trash(n_m), n_c + _trash(n_c)
    f = pl.kernel(
        functools.partial(_sc_counts_body, n_p, n_m, n_c),
        compiler_params=pltpu.CompilerParams(use_tc_tiling_on_sc=False),
        out_type=[jax.ShapeDtypeStruct((nm_f,), jnp.float32),
                  jax.ShapeDtypeStruct((np_f,), jnp.float32),
                  jax.ShapeDtypeStruct((nc_f,), jnp.float32),
                  jax.ShapeDtypeStruct((np_f,), jnp.float32)],
        mesh=mesh,
        scratch_types=[
            pltpu.VMEM_SHARED((np_f,), jnp.float32),
            pltpu.VMEM_SHARED((nm_f,), jnp.float32),
            pltpu.VMEM((8, 128), jnp.int32),
            pltpu.VMEM((128,), jnp.float32),
            pltpu.SemaphoreType.DMA,
            pltpu.VMEM(((n_p + _trash(n_p)) // _NT,), jnp.float32),
        ])
    return f(dpm, dmp, dpc, dcp, zeros1)


def _pad_edges(e, n_src, n_dst, e_pad):
    pad = e_pad - e.shape[1]
    idx = jnp.arange(pad, dtype=jnp.int32)
    ext = jnp.concatenate(
        [e, jnp.stack([idx % n_src, n_dst + (idx % _trash(n_dst))])], axis=1)
    return ext[0].reshape(-1, 128), ext[1].reshape(-1, 128)


def kernel(x_patient, x_med, x_com, edge_pm, edge_mp, edge_pc, edge_cp,
           proj_Wp, proj_bp, proj_Wm, proj_bm, proj_Wc, proj_bc,
           Wl, bl, Wr, cls_W, cls_b):
    n_p, n_m, n_c = x_patient.shape[0], x_med.shape[0], x_com.shape[0]
    EPM_PAD = _NT * 196 * _KR * 128
    EPC_PAD = _NT * 52 * _KR * 128

    spm, dpm = _pad_edges(edge_pm, n_p, n_m, EPM_PAD)
    smp, dmp = _pad_edges(edge_mp, n_m, n_p, EPM_PAD)
    spc, dpc = _pad_edges(edge_pc, n_p, n_c, EPC_PAD)
    scp, dcp = _pad_edges(edge_cp, n_c, n_p, EPC_PAD)
    eidx = (spm, dpm, smp, dmp, spc, dpc, scp, dcp)
    zeros2d = jnp.zeros((n_p + _trash(n_p), HH), jnp.float32)
    zeros1 = jnp.zeros((n_p + _trash(n_p),), jnp.float32)

    xp = _proj(x_patient, proj_Wp.T, proj_bp)
    xm = _proj(x_med, proj_Wm.T, proj_bm)
    xc = _proj(x_com, proj_Wc.T, proj_bc)

    cnt_m, cnt_p1, cnt_c, cnt_p2 = [
        a.reshape(-1, 1)
        for a in _sc_counts(dpm, dmp, dpc, dcp, zeros1, n_p, n_m, n_c)]

    s_pm, s_mp, s_pc, s_cp = _sc_agg(xp, xm, xc, eidx, zeros2d, n_p, n_m, n_c)
    xm1 = _comb1(s_pm, cnt_m, xm, Wl[0, 0].T, Wr[0, 0].T, bl[0, 0],
                 relu=True, n=n_m)
    xc1 = _comb1(s_pc, cnt_c, xc, Wl[0, 2].T, Wr[0, 2].T, bl[0, 2],
                 relu=True, n=n_c)
    xp1 = _comb2(s_mp, cnt_p1, s_cp, cnt_p2, xp, Wl[0, 1].T, Wl[0, 3].T,
                 (Wr[0, 1] + Wr[0, 3]).T, bl[0, 1] + bl[0, 3],
                 relu=True, n=n_p)

    s_pm2, s_mp2, s_pc2, s_cp2 = _sc_agg(xp1, xm1, xc1, eidx, zeros2d,
                                         n_p, n_m, n_c)

    cm_mean_m, cm_x_m = _colmean(s_pm2, cnt_m, xm1, n=n_m)
    cm_mean_c, cm_x_c = _colmean(s_pc2, cnt_c, xc1, n=n_c)

    w1, w2, w3 = cls_W[0, :H], cls_W[0, H:2 * H], cls_W[0, 2 * H:]
    a1 = Wl[1, 1].T @ w1
    a2 = Wl[1, 3].T @ w1
    av = (Wr[1, 1] + Wr[1, 3]).T @ w1
    g1, g2 = Wl[1, 0].T @ w2, Wr[1, 0].T @ w2
    h1, h2 = Wl[1, 2].T @ w3, Wr[1, 2].T @ w3
    s_scalar = (cm_mean_m[0] @ g1 + cm_x_m[0] @ g2
                + cm_mean_c[0] @ h1 + cm_x_c[0] @ h2
                + (bl[1, 1] + bl[1, 3]) @ w1 + bl[1, 0] @ w2 + bl[1, 2] @ w3
                + cls_b[0])
    amat = jnp.stack([a1, a2, av], axis=1)

    return _final(s_mp2, cnt_p1, s_cp2, cnt_p2, xp1, amat,
                  s_scalar.reshape(1, 1), n=n_p)


# speedup vs baseline: 12.4396x; 1.1114x over previous
import functools

import jax
import jax.numpy as jnp
from jax import lax
from jax.experimental import pallas as pl
from jax.experimental.pallas import tpu as pltpu, tpu_sc as plsc

H = 64
HH = 32
_PREC = lax.Precision.HIGHEST
_NT = 16


def _trash(n):
    t = (-n) % 128
    return t if t else 128


def _dot(a, b):
    return jnp.dot(a, b, preferred_element_type=jnp.float32, precision=_PREC)


def _proj_body(x_ref, w_ref, b_ref, o_ref):
    y = _dot(x_ref[...], w_ref[...]) + b_ref[...]
    o_ref[0] = y[:, :HH]
    o_ref[1] = y[:, HH:]


def _proj(x, wT, b, R=1000):
    n, d = x.shape
    return pl.pallas_call(
        _proj_body,
        grid=(n // R,),
        in_specs=[pl.BlockSpec((R, d), lambda i: (i, 0)),
                  pl.BlockSpec((d, H), lambda i: (0, 0)),
                  pl.BlockSpec((1, H), lambda i: (0, 0))],
        out_specs=pl.BlockSpec((2, R, HH), lambda i: (0, i, 0)),
        out_shape=jax.ShapeDtypeStruct((2, n + _trash(n), HH), jnp.float32),
    )(x, wT, b.reshape(1, H))


def _cat(ref):
    return jnp.concatenate([ref[0], ref[1]], axis=1)


def _mean(s_ref, cnt_ref):
    return _cat(s_ref) * (1.0 / jnp.maximum(cnt_ref[...], 1.0))


def _comb1_body(relu, s_ref, cnt_ref, x_ref, a_ref, bmat_ref, bias_ref, o_ref):
    y = _dot(_mean(s_ref, cnt_ref), a_ref[...]) + _dot(_cat(x_ref), bmat_ref[...]) + bias_ref[...]
    if relu:
        y = jnp.maximum(y, 0.0)
    o_ref[0] = y[:, :HH]
    o_ref[1] = y[:, HH:]


def _comb1(s, cnt, x, aT, bT, bias, relu, n, R=1000):
    return pl.pallas_call(
        functools.partial(_comb1_body, relu),
        grid=(n // R,),
        in_specs=[pl.BlockSpec((2, R, HH), lambda i: (0, i, 0)),
                  pl.BlockSpec((R, 1), lambda i: (i, 0)),
                  pl.BlockSpec((2, R, HH), lambda i: (0, i, 0)),
                  pl.BlockSpec((H, H), lambda i: (0, 0)),
                  pl.BlockSpec((H, H), lambda i: (0, 0)),
                  pl.BlockSpec((1, H), lambda i: (0, 0))],
        out_specs=pl.BlockSpec((2, R, HH), lambda i: (0, i, 0)),
        out_shape=jax.ShapeDtypeStruct((2, n + _trash(n), HH), jnp.float32),
    )(s, cnt, x, aT, bT, bias.reshape(1, H))


def _comb2_body(relu, s1_ref, c1_ref, s2_ref, c2_ref, x_ref, a1_ref, a2_ref,
                bmat_ref, bias_ref, o_ref):
    y = (_dot(_mean(s1_ref, c1_ref), a1_ref[...])
         + _dot(_mean(s2_ref, c2_ref), a2_ref[...])
         + _dot(_cat(x_ref), bmat_ref[...]) + bias_ref[...])
    if relu:
        y = jnp.maximum(y, 0.0)
    o_ref[0] = y[:, :HH]
    o_ref[1] = y[:, HH:]


def _comb2(s1, c1, s2, c2, x, a1T, a2T, bT, bias, relu, n, R=1000):
    return pl.pallas_call(
        functools.partial(_comb2_body, relu),
        grid=(n // R,),
        in_specs=[pl.BlockSpec((2, R, HH), lambda i: (0, i, 0)),
                  pl.BlockSpec((R, 1), lambda i: (i, 0)),
                  pl.BlockSpec((2, R, HH), lambda i: (0, i, 0)),
                  pl.BlockSpec((R, 1), lambda i: (i, 0)),
                  pl.BlockSpec((2, R, HH), lambda i: (0, i, 0)),
                  pl.BlockSpec((H, H), lambda i: (0, 0)),
                  pl.BlockSpec((H, H), lambda i: (0, 0)),
                  pl.BlockSpec((H, H), lambda i: (0, 0)),
                  pl.BlockSpec((1, H), lambda i: (0, 0))],
        out_specs=pl.BlockSpec((2, R, HH), lambda i: (0, i, 0)),
        out_shape=jax.ShapeDtypeStruct((2, n + _trash(n), HH), jnp.float32),
    )(s1, c1, s2, c2, x, a1T, a2T, bT, bias.reshape(1, H))


def _colmean_body(inv_n, s_ref, cnt_ref, x_ref, o1_ref, o2_ref):
    @pl.when(pl.program_id(0) == 0)
    def _():
        o1_ref[...] = jnp.zeros_like(o1_ref)
        o2_ref[...] = jnp.zeros_like(o2_ref)

    o1_ref[...] += jnp.sum(_mean(s_ref, cnt_ref), axis=0, keepdims=True) * inv_n
    o2_ref[...] += jnp.sum(_cat(x_ref), axis=0, keepdims=True) * inv_n


def _colmean(s, cnt, x, n, R=1000):
    return pl.pallas_call(
        functools.partial(_colmean_body, 1.0 / n),
        grid=(n // R,),
        in_specs=[pl.BlockSpec((2, R, HH), lambda i: (0, i, 0)),
                  pl.BlockSpec((R, 1), lambda i: (i, 0)),
                  pl.BlockSpec((2, R, HH), lambda i: (0, i, 0))],
        out_specs=[pl.BlockSpec((1, H), lambda i: (0, 0)),
                   pl.BlockSpec((1, H), lambda i: (0, 0))],
        out_shape=[jax.ShapeDtypeStruct((1, H), jnp.float32),
                   jax.ShapeDtypeStruct((1, H), jnp.float32)],
    )(s, cnt, x)


def _final_body(s1_ref, c1_ref, s2_ref, c2_ref, x_ref, amat_ref, const_ref, o_ref):
    y = (_dot(_mean(s1_ref, c1_ref), amat_ref[:, 0:1])
         + _dot(_mean(s2_ref, c2_ref), amat_ref[:, 1:2])
         + _dot(_cat(x_ref), amat_ref[:, 2:3]) + const_ref[...])
    o_ref[...] = y


def _final(s1, c1, s2, c2, x, amat, const, n, R=1000):
    return pl.pallas_call(
        _final_body,
        grid=(n // R,),
        in_specs=[pl.BlockSpec((2, R, HH), lambda i: (0, i, 0)),
                  pl.BlockSpec((R, 1), lambda i: (i, 0)),
                  pl.BlockSpec((2, R, HH), lambda i: (0, i, 0)),
                  pl.BlockSpec((R, 1), lambda i: (i, 0)),
                  pl.BlockSpec((2, R, HH), lambda i: (0, i, 0)),
                  pl.BlockSpec((H, 3), lambda i: (0, 0)),
                  pl.BlockSpec((1, 1), lambda i: (0, 0))],
        out_specs=pl.BlockSpec((R, 1), lambda i: (i, 0)),
        out_shape=jax.ShapeDtypeStruct((n, 1), jnp.float32),
    )(s1, c1, s2, c2, x, amat, const)


_KR = 2


def _sc_agg_body(n_p, n_m, n_c, xp_s, xm_s, xc_s, spm, dpm, smp, dmp, spc,
                 dpc, scp, dcp, zeros2d, o_pm, o_mp, o_pc, o_cp,
                 acc, gbuf, dbuf, rows, si0, si1, si2, si3, sg0, sg1, ss0, ss1):
    semi = (si0, si1, si2, si3)
    semg = (sg0, sg1)
    sems = (ss0, ss1)
    c = lax.axis_index("c")
    t = lax.axis_index("s")
    np_f, nm_f, nc_f = n_p + _trash(n_p), n_m + _trash(n_m), n_c + _trash(n_c)

    def zero(n):
        k = n // _NT
        pltpu.sync_copy(zeros2d.at[pl.ds(t * k, k)], acc.at[pl.ds(t * k, k)])

    def writeback(n, out):
        k = n // _NT
        pltpu.sync_copy(acc.at[pl.ds(t * k, k)], out.at[c, pl.ds(t * k, k)])

    def edge_loop(s2d, d2d, tbl):
        rounds = s2d.shape[0] // (_NT * _KR)
        quads = rounds // 4
        base = t * rounds * _KR

        def fire_idx(g, sl):
            r0 = base + g * _KR
            pltpu.async_copy(s2d.at[pl.ds(r0, _KR)], gbuf.at[sl], semi[sl])
            pltpu.async_copy(d2d.at[pl.ds(r0, _KR)], dbuf.at[sl], semi[sl])

        def wait_idx(sl):
            pltpu.make_async_copy(s2d.at[pl.ds(base, _KR)], gbuf.at[sl], semi[sl]).wait()
            pltpu.make_async_copy(d2d.at[pl.ds(base, _KR)], dbuf.at[sl], semi[sl]).wait()

        def fire_g(isl, rsl):
            for j in range(_KR):
                pltpu.async_copy(tbl.at[gbuf.at[isl, j]],
                                 rows.at[rsl, pl.ds(j * 128, 128)], semg[rsl])

        def wait_g(rsl):
            for j in range(_KR):
                pltpu.make_async_copy(tbl.at[gbuf.at[0, j]],
                                      rows.at[rsl, pl.ds(j * 128, 128)], semg[rsl]).wait()

        def fire_s(isl, rsl):
            for j in range(_KR):
                pltpu.async_copy(rows.at[rsl, pl.ds(j * 128, 128)],
                                 acc.at[dbuf.at[isl, j]], sems[rsl], add=True)

        def wait_s(rsl):
            for j in range(_KR):
                pltpu.make_async_copy(rows.at[rsl, pl.ds(j * 128, 128)],
                                      acc.at[dbuf.at[0, j]], sems[rsl]).wait()

        fire_idx(0, 0)
        fire_idx(1, 1)

        def body(q, carry):
            g0 = 4 * q
            wait_idx(0)

            @pl.when(q > 0)
            def _():
                wait_s(0)

            fire_g(0, 0)
            wait_idx(1)

            @pl.when(q > 0)
            def _():
                wait_s(1)

            fire_g(1, 1)
            fire_idx(g0 + 2, 2)
            fire_idx(g0 + 3, 3)
            wait_g(0)
            fire_s(0, 0)
            wait_g(1)
            fire_s(1, 1)
            wait_idx(2)
            wait_s(0)
            fire_g(2, 0)
            wait_idx(3)
            wait_s(1)
            fire_g(3, 1)

            @pl.when(q + 1 < quads)
            def _():
                fire_idx(g0 + 4, 0)
                fire_idx(g0 + 5, 1)

            wait_g(0)
            fire_s(2, 0)
            wait_g(1)
            fire_s(3, 1)
            return carry

        lax.fori_loop(0, quads, body, 0)
        wait_s(0)
        wait_s(1)

    bar = plsc.subcore_barrier
    for src2d, dst2d, x_hbm, n_out, out in (
            (spm, dpm, xp_s, nm_f, o_pm),
            (spc, dpc, xp_s, nc_f, o_pc),
            (smp, dmp, xm_s, np_f, o_mp),
            (scp, dcp, xc_s, np_f, o_cp)):
        zero(n_out)
        bar()
        edge_loop(src2d, dst2d, x_hbm.at[c])
        bar()
        writeback(n_out, out)
        bar()


def _sc_agg(xp_s, xm_s, xc_s, eidx, zeros2d, n_p, n_m, n_c):
    mesh = plsc.VectorSubcoreMesh(core_axis_name="c", subcore_axis_name="s")
    np_f, nm_f, nc_f = n_p + _trash(n_p), n_m + _trash(n_m), n_c + _trash(n_c)
    f = pl.kernel(
        functools.partial(_sc_agg_body, n_p, n_m, n_c),
        compiler_params=pltpu.CompilerParams(use_tc_tiling_on_sc=False),
        out_type=[jax.ShapeDtypeStruct((2, nm_f, HH), jnp.float32),
                  jax.ShapeDtypeStruct((2, np_f, HH), jnp.float32),
                  jax.ShapeDtypeStruct((2, nc_f, HH), jnp.float32),
                  jax.ShapeDtypeStruct((2, np_f, HH), jnp.float32)],
        mesh=mesh,
        scratch_types=[
            pltpu.VMEM_SHARED((np_f, HH), jnp.float32),
            pltpu.VMEM((4, _KR, 128), jnp.int32),
            pltpu.VMEM((4, _KR, 128), jnp.int32),
            pltpu.VMEM((2, _KR * 128, HH), jnp.float32),
        ] + [pltpu.SemaphoreType.DMA] * 8)
    import os as _os
    if _os.environ.get("SCSTUB"):
        np_f2, nm_f2, nc_f2 = n_p + _trash(n_p), n_m + _trash(n_m), n_c + _trash(n_c)
        z = lambda n: jnp.zeros((2, n, HH), jnp.float32) + xp_s[0, 0, 0]
        return z(nm_f2), z(np_f2), z(nc_f2), z(np_f2)
    return f(xp_s, xm_s, xc_s, *eidx, zeros2d)


def _sc_counts_body(n_p, n_m, n_c, dpm, dmp, dpc, dcp, zeros1,
                    o_m, o_p1, o_c, o_p2, acc_a, acc_b, dbuf, ones, sem, b1d):
    c = lax.axis_index("c")
    t = lax.axis_index("s")
    np_f, nm_f, nc_f = n_p + _trash(n_p), n_m + _trash(n_m), n_c + _trash(n_c)

    for i in range(8):
        ones[pl.ds(i * 16, 16)] = jnp.ones((16,), jnp.float32)

    def zero1(acc, n):
        k = n // _NT
        pltpu.sync_copy(zeros1.at[pl.ds(t * k, k)], b1d.at[pl.ds(0, k)])
        pltpu.sync_copy(b1d.at[pl.ds(0, k)], acc.at[pl.ds(t * k, k)])

    def wb1(acc, n, out):
        k = n // _NT
        pltpu.sync_copy(acc.at[pl.ds(t * k, k)], b1d.at[pl.ds(0, k)])
        pltpu.sync_copy(b1d.at[pl.ds(0, k)], out.at[pl.ds(t * k, k)])

    def cnt_loop(d2d, kr, acc):
        rounds = d2d.shape[0] // (_NT * kr)
        base = t * rounds * kr

        def body(g, carry):
            r0 = base + g * kr
            pltpu.sync_copy(d2d.at[pl.ds(r0, kr)], dbuf.at[pl.ds(0, kr)])
            ds_ = [pltpu.async_copy(ones, acc.at[dbuf.at[j]], sem, add=True)
                   for j in range(kr)]
            for dd in ds_:
                dd.wait()
            return carry

        lax.fori_loop(0, rounds, body, 0)

    bar = plsc.subcore_barrier
    zero1(acc_a, np_f)
    zero1(acc_b, nm_f)
    bar()

    @pl.when(c == 0)
    def _():
        cnt_loop(dpm, 8, acc_b)
        cnt_loop(dmp, 8, acc_a)

    @pl.when(c == 1)
    def _():
        cnt_loop(dpc, 2, acc_b)
        cnt_loop(dcp, 2, acc_a)

    bar()

    @pl.when(c == 0)
    def _():
        wb1(acc_b, nm_f, o_m)
        wb1(acc_a, np_f, o_p1)

    @pl.when(c == 1)
    def _():
        wb1(acc_b, nc_f, o_c)
        wb1(acc_a, np_f, o_p2)


def _sc_counts(dpm, dmp, dpc, dcp, zeros1, n_p, n_m, n_c):
    mesh = plsc.VectorSubcoreMesh(core_axis_name="c", subcore_axis_name="s")
    np_f, nm_f, nc_f = n_p + _trash(n_p), n_m + _trash(n_m), n_c + _trash(n_c)
    f = pl.kernel(
        functools.partial(_sc_counts_body, n_p, n_m, n_c),
        compiler_params=pltpu.CompilerParams(use_tc_tiling_on_sc=False),
        out_type=[jax.ShapeDtypeStruct((nm_f,), jnp.float32),
                  jax.ShapeDtypeStruct((np_f,), jnp.float32),
                  jax.ShapeDtypeStruct((nc_f,), jnp.float32),
                  jax.ShapeDtypeStruct((np_f,), jnp.float32)],
        mesh=mesh,
        scratch_types=[
            pltpu.VMEM_SHARED((np_f,), jnp.float32),
            pltpu.VMEM_SHARED((nm_f,), jnp.float32),
            pltpu.VMEM((8, 128), jnp.int32),
            pltpu.VMEM((128,), jnp.float32),
            pltpu.SemaphoreType.DMA,
            pltpu.VMEM(((n_p + _trash(n_p)) // _NT,), jnp.float32),
        ])
    return f(dpm, dmp, dpc, dcp, zeros1)


def _pad_edges(e, n_src, n_dst, e_pad):
    pad = e_pad - e.shape[1]
    idx = jnp.arange(pad, dtype=jnp.int32)
    ext = jnp.concatenate(
        [e, jnp.stack([idx % n_src, n_dst + (idx % _trash(n_dst))])], axis=1)
    return ext[0].reshape(-1, 128), ext[1].reshape(-1, 128)


def kernel(x_patient, x_med, x_com, edge_pm, edge_mp, edge_pc, edge_cp,
           proj_Wp, proj_bp, proj_Wm, proj_bm, proj_Wc, proj_bc,
           Wl, bl, Wr, cls_W, cls_b):
    n_p, n_m, n_c = x_patient.shape[0], x_med.shape[0], x_com.shape[0]
    EPM_PAD = _NT * 196 * _KR * 128
    EPC_PAD = _NT * 52 * _KR * 128

    spm, dpm = _pad_edges(edge_pm, n_p, n_m, EPM_PAD)
    smp, dmp = _pad_edges(edge_mp, n_m, n_p, EPM_PAD)
    spc, dpc = _pad_edges(edge_pc, n_p, n_c, EPC_PAD)
    scp, dcp = _pad_edges(edge_cp, n_c, n_p, EPC_PAD)
    eidx = (spm, dpm, smp, dmp, spc, dpc, scp, dcp)
    zeros2d = jnp.zeros((n_p + _trash(n_p), HH), jnp.float32)
    zeros1 = jnp.zeros((n_p + _trash(n_p),), jnp.float32)

    xp = _proj(x_patient, proj_Wp.T, proj_bp)
    xm = _proj(x_med, proj_Wm.T, proj_bm)
    xc = _proj(x_com, proj_Wc.T, proj_bc)

    cnt_m, cnt_p1, cnt_c, cnt_p2 = [
        a.reshape(-1, 1)
        for a in _sc_counts(dpm, dmp, dpc, dcp, zeros1, n_p, n_m, n_c)]

    s_pm, s_mp, s_pc, s_cp = _sc_agg(xp, xm, xc, eidx, zeros2d, n_p, n_m, n_c)
    xm1 = _comb1(s_pm, cnt_m, xm, Wl[0, 0].T, Wr[0, 0].T, bl[0, 0],
                 relu=True, n=n_m)
    xc1 = _comb1(s_pc, cnt_c, xc, Wl[0, 2].T, Wr[0, 2].T, bl[0, 2],
                 relu=True, n=n_c)
    xp1 = _comb2(s_mp, cnt_p1, s_cp, cnt_p2, xp, Wl[0, 1].T, Wl[0, 3].T,
                 (Wr[0, 1] + Wr[0, 3]).T, bl[0, 1] + bl[0, 3],
                 relu=True, n=n_p)

    s_pm2, s_mp2, s_pc2, s_cp2 = _sc_agg(xp1, xm1, xc1, eidx, zeros2d,
                                         n_p, n_m, n_c)

    cm_mean_m, cm_x_m = _colmean(s_pm2, cnt_m, xm1, n=n_m)
    cm_mean_c, cm_x_c = _colmean(s_pc2, cnt_c, xc1, n=n_c)

    w1, w2, w3 = cls_W[0, :H], cls_W[0, H:2 * H], cls_W[0, 2 * H:]
    a1 = Wl[1, 1].T @ w1
    a2 = Wl[1, 3].T @ w1
    av = (Wr[1, 1] + Wr[1, 3]).T @ w1
    g1, g2 = Wl[1, 0].T @ w2, Wr[1, 0].T @ w2
    h1, h2 = Wl[1, 2].T @ w3, Wr[1, 2].T @ w3
    s_scalar = (cm_mean_m[0] @ g1 + cm_x_m[0] @ g2
                + cm_mean_c[0] @ h1 + cm_x_c[0] @ h2
                + (bl[1, 1] + bl[1, 3]) @ w1 + bl[1, 0] @ w2 + bl[1, 2] @ w3
                + cls_b[0])
    amat = jnp.stack([a1, a2, av], axis=1)

    return _final(s_mp2, cnt_p1, s_cp2, cnt_p2, xp1, amat,
                  s_scalar.reshape(1, 1), n=n_p)


# speedup vs baseline: 14.3179x; 1.1510x over previous
import functools

import jax
import jax.numpy as jnp
from jax import lax
from jax.experimental import pallas as pl
from jax.experimental.pallas import tpu as pltpu, tpu_sc as plsc

H = 64
HH = 32
_PREC = lax.Precision.HIGHEST
_NT = 16


def _trash(n):
    t = (-n) % 128
    return t if t else 128


def _dot(a, b):
    return jnp.dot(a, b, preferred_element_type=jnp.float32, precision=_PREC)


def _proj_body(x_ref, w_ref, b_ref, o_ref):
    y = _dot(x_ref[...], w_ref[...]) + b_ref[...]
    o_ref[0] = y[:, :HH]
    o_ref[1] = y[:, HH:]


def _proj(x, wT, b, R=1000):
    n, d = x.shape
    return pl.pallas_call(
        _proj_body,
        grid=(n // R,),
        in_specs=[pl.BlockSpec((R, d), lambda i: (i, 0)),
                  pl.BlockSpec((d, H), lambda i: (0, 0)),
                  pl.BlockSpec((1, H), lambda i: (0, 0))],
        out_specs=pl.BlockSpec((2, R, HH), lambda i: (0, i, 0)),
        out_shape=jax.ShapeDtypeStruct((2, n + _trash(n), HH), jnp.float32),
    )(x, wT, b.reshape(1, H))


def _cat(ref):
    return jnp.concatenate([ref[0], ref[1]], axis=1)


def _mean(s_ref, cnt_ref):
    return _cat(s_ref) * (1.0 / jnp.maximum(cnt_ref[...], 1.0))


def _comb1_body(relu, s_ref, cnt_ref, x_ref, a_ref, bmat_ref, bias_ref, o_ref):
    y = _dot(_mean(s_ref, cnt_ref), a_ref[...]) + _dot(_cat(x_ref), bmat_ref[...]) + bias_ref[...]
    if relu:
        y = jnp.maximum(y, 0.0)
    o_ref[0] = y[:, :HH]
    o_ref[1] = y[:, HH:]


def _comb1(s, cnt, x, aT, bT, bias, relu, n, R=1000):
    return pl.pallas_call(
        functools.partial(_comb1_body, relu),
        grid=(n // R,),
        in_specs=[pl.BlockSpec((2, R, HH), lambda i: (0, i, 0)),
                  pl.BlockSpec((R, 1), lambda i: (i, 0)),
                  pl.BlockSpec((2, R, HH), lambda i: (0, i, 0)),
                  pl.BlockSpec((H, H), lambda i: (0, 0)),
                  pl.BlockSpec((H, H), lambda i: (0, 0)),
                  pl.BlockSpec((1, H), lambda i: (0, 0))],
        out_specs=pl.BlockSpec((2, R, HH), lambda i: (0, i, 0)),
        out_shape=jax.ShapeDtypeStruct((2, n + _trash(n), HH), jnp.float32),
    )(s, cnt, x, aT, bT, bias.reshape(1, H))


def _comb2_body(relu, s1_ref, c1_ref, s2_ref, c2_ref, x_ref, a1_ref, a2_ref,
                bmat_ref, bias_ref, o_ref):
    y = (_dot(_mean(s1_ref, c1_ref), a1_ref[...])
         + _dot(_mean(s2_ref, c2_ref), a2_ref[...])
         + _dot(_cat(x_ref), bmat_ref[...]) + bias_ref[...])
    if relu:
        y = jnp.maximum(y, 0.0)
    o_ref[0] = y[:, :HH]
    o_ref[1] = y[:, HH:]


def _comb2(s1, c1, s2, c2, x, a1T, a2T, bT, bias, relu, n, R=1000):
    return pl.pallas_call(
        functools.partial(_comb2_body, relu),
        grid=(n // R,),
        in_specs=[pl.BlockSpec((2, R, HH), lambda i: (0, i, 0)),
                  pl.BlockSpec((R, 1), lambda i: (i, 0)),
                  pl.BlockSpec((2, R, HH), lambda i: (0, i, 0)),
                  pl.BlockSpec((R, 1), lambda i: (i, 0)),
                  pl.BlockSpec((2, R, HH), lambda i: (0, i, 0)),
                  pl.BlockSpec((H, H), lambda i: (0, 0)),
                  pl.BlockSpec((H, H), lambda i: (0, 0)),
                  pl.BlockSpec((H, H), lambda i: (0, 0)),
                  pl.BlockSpec((1, H), lambda i: (0, 0))],
        out_specs=pl.BlockSpec((2, R, HH), lambda i: (0, i, 0)),
        out_shape=jax.ShapeDtypeStruct((2, n + _trash(n), HH), jnp.float32),
    )(s1, c1, s2, c2, x, a1T, a2T, bT, bias.reshape(1, H))


def _colmean_body(inv_n, s_ref, cnt_ref, x_ref, o1_ref, o2_ref):
    @pl.when(pl.program_id(0) == 0)
    def _():
        o1_ref[...] = jnp.zeros_like(o1_ref)
        o2_ref[...] = jnp.zeros_like(o2_ref)

    o1_ref[...] += jnp.sum(_mean(s_ref, cnt_ref), axis=0, keepdims=True) * inv_n
    o2_ref[...] += jnp.sum(_cat(x_ref), axis=0, keepdims=True) * inv_n


def _colmean(s, cnt, x, n, R=1000):
    return pl.pallas_call(
        functools.partial(_colmean_body, 1.0 / n),
        grid=(n // R,),
        in_specs=[pl.BlockSpec((2, R, HH), lambda i: (0, i, 0)),
                  pl.BlockSpec((R, 1), lambda i: (i, 0)),
                  pl.BlockSpec((2, R, HH), lambda i: (0, i, 0))],
        out_specs=[pl.BlockSpec((1, H), lambda i: (0, 0)),
                   pl.BlockSpec((1, H), lambda i: (0, 0))],
        out_shape=[jax.ShapeDtypeStruct((1, H), jnp.float32),
                   jax.ShapeDtypeStruct((1, H), jnp.float32)],
    )(s, cnt, x)


def _final_body(s1_ref, c1_ref, s2_ref, c2_ref, x_ref, amat_ref, const_ref, o_ref):
    y = (_dot(_mean(s1_ref, c1_ref), amat_ref[:, 0:1])
         + _dot(_mean(s2_ref, c2_ref), amat_ref[:, 1:2])
         + _dot(_cat(x_ref), amat_ref[:, 2:3]) + const_ref[...])
    o_ref[...] = y


def _final(s1, c1, s2, c2, x, amat, const, n, R=1000):
    return pl.pallas_call(
        _final_body,
        grid=(n // R,),
        in_specs=[pl.BlockSpec((2, R, HH), lambda i: (0, i, 0)),
                  pl.BlockSpec((R, 1), lambda i: (i, 0)),
                  pl.BlockSpec((2, R, HH), lambda i: (0, i, 0)),
                  pl.BlockSpec((R, 1), lambda i: (i, 0)),
                  pl.BlockSpec((2, R, HH), lambda i: (0, i, 0)),
                  pl.BlockSpec((H, 3), lambda i: (0, 0)),
                  pl.BlockSpec((1, 1), lambda i: (0, 0))],
        out_specs=pl.BlockSpec((R, 1), lambda i: (i, 0)),
        out_shape=jax.ShapeDtypeStruct((n, 1), jnp.float32),
    )(s1, c1, s2, c2, x, amat, const)


_KR = 2


def _sc_agg_body(n_p, n_m, n_c, xp_s, xm_s, xc_s, spm, dpm, smp, dmp, spc,
                 dpc, scp, dcp, zeros2d, o_pm, o_mp, o_pc, o_cp,
                 acc, gbuf, dbuf, rows, *sems16):
    semi = sems16[:8]
    semg = sems16[8:12]
    sems = sems16[12:16]
    c = lax.axis_index("c")
    t = lax.axis_index("s")
    np_f, nm_f, nc_f = n_p + _trash(n_p), n_m + _trash(n_m), n_c + _trash(n_c)

    def zero(n):
        k = n // _NT
        pltpu.sync_copy(zeros2d.at[pl.ds(t * k, k)], acc.at[pl.ds(t * k, k)])

    def writeback(n, out):
        k = n // _NT
        pltpu.sync_copy(acc.at[pl.ds(t * k, k)], out.at[c, pl.ds(t * k, k)])

    def edge_loop(s2d, d2d, tbl):
        rounds = s2d.shape[0] // _NT
        octs = rounds // 8
        base = t * rounds

        def fire_idx(g, sl):
            r0 = base + g
            pltpu.async_copy(s2d.at[r0], gbuf.at[sl], semi[sl])
            pltpu.async_copy(d2d.at[r0], dbuf.at[sl], semi[sl])

        def wait_idx(sl):
            pltpu.make_async_copy(s2d.at[0], gbuf.at[sl], semi[sl]).wait()
            pltpu.make_async_copy(d2d.at[0], dbuf.at[sl], semi[sl]).wait()

        def fire_g(isl, rsl):
            pltpu.async_copy(tbl.at[gbuf.at[isl]], rows.at[rsl], semg[rsl])

        def wait_g(rsl):
            pltpu.make_async_copy(tbl.at[gbuf.at[0]], rows.at[rsl], semg[rsl]).wait()

        def fire_s(isl, rsl):
            pltpu.async_copy(rows.at[rsl], acc.at[dbuf.at[isl]], sems[rsl], add=True)

        def wait_s(rsl):
            pltpu.make_async_copy(rows.at[rsl], acc.at[dbuf.at[0]], sems[rsl]).wait()

        for sl in range(4):
            fire_idx(sl, sl)

        def body(o, carry):
            g0 = 8 * o
            for k in range(4):
                wait_idx(k)

                @pl.when(o > 0)
                def _():
                    wait_s(k)

                fire_g(k, k)
            for k in range(4):
                fire_idx(g0 + 4 + k, 4 + k)
            for k in range(4):
                wait_g(k)
                fire_s(k, k)
            for k in range(4):
                wait_idx(4 + k)
                wait_s(k)
                fire_g(4 + k, k)

            @pl.when(o + 1 < octs)
            def _():
                for k in range(4):
                    fire_idx(g0 + 8 + k, k)

            for k in range(4):
                wait_g(k)
                fire_s(4 + k, k)
            return carry

        lax.fori_loop(0, octs, body, 0)
        for k in range(4):
            wait_s(k)

    bar = plsc.subcore_barrier
    for src2d, dst2d, x_hbm, n_out, out in (
            (spm, dpm, xp_s, nm_f, o_pm),
            (spc, dpc, xp_s, nc_f, o_pc),
            (smp, dmp, xm_s, np_f, o_mp),
            (scp, dcp, xc_s, np_f, o_cp)):
        zero(n_out)
        bar()
        edge_loop(src2d, dst2d, x_hbm.at[c])
        bar()
        writeback(n_out, out)
        bar()


def _sc_agg(xp_s, xm_s, xc_s, eidx, zeros2d, n_p, n_m, n_c):
    mesh = plsc.VectorSubcoreMesh(core_axis_name="c", subcore_axis_name="s")
    np_f, nm_f, nc_f = n_p + _trash(n_p), n_m + _trash(n_m), n_c + _trash(n_c)
    f = pl.kernel(
        functools.partial(_sc_agg_body, n_p, n_m, n_c),
        compiler_params=pltpu.CompilerParams(use_tc_tiling_on_sc=False),
        out_type=[jax.ShapeDtypeStruct((2, nm_f, HH), jnp.float32),
                  jax.ShapeDtypeStruct((2, np_f, HH), jnp.float32),
                  jax.ShapeDtypeStruct((2, nc_f, HH), jnp.float32),
                  jax.ShapeDtypeStruct((2, np_f, HH), jnp.float32)],
        mesh=mesh,
        scratch_types=[
            pltpu.VMEM_SHARED((np_f, HH), jnp.float32),
            pltpu.VMEM((8, 128), jnp.int32),
            pltpu.VMEM((8, 128), jnp.int32),
            pltpu.VMEM((4, 128, HH), jnp.float32),
        ] + [pltpu.SemaphoreType.DMA] * 16)
    import os as _os
    if _os.environ.get("SCSTUB"):
        np_f2, nm_f2, nc_f2 = n_p + _trash(n_p), n_m + _trash(n_m), n_c + _trash(n_c)
        z = lambda n: jnp.zeros((2, n, HH), jnp.float32) + xp_s[0, 0, 0]
        return z(nm_f2), z(np_f2), z(nc_f2), z(np_f2)
    return f(xp_s, xm_s, xc_s, *eidx, zeros2d)


def _sc_counts_body(n_p, n_m, n_c, dpm, dmp, dpc, dcp, zeros1,
                    o_m, o_p1, o_c, o_p2, acc_a, acc_b, dbuf, ones, sem, b1d):
    c = lax.axis_index("c")
    t = lax.axis_index("s")
    np_f, nm_f, nc_f = n_p + _trash(n_p), n_m + _trash(n_m), n_c + _trash(n_c)

    for i in range(8):
        ones[pl.ds(i * 16, 16)] = jnp.ones((16,), jnp.float32)

    def zero1(acc, n):
        k = n // _NT
        pltpu.sync_copy(zeros1.at[pl.ds(t * k, k)], b1d.at[pl.ds(0, k)])
        pltpu.sync_copy(b1d.at[pl.ds(0, k)], acc.at[pl.ds(t * k, k)])

    def wb1(acc, n, out):
        k = n // _NT
        pltpu.sync_copy(acc.at[pl.ds(t * k, k)], b1d.at[pl.ds(0, k)])
        pltpu.sync_copy(b1d.at[pl.ds(0, k)], out.at[pl.ds(t * k, k)])

    def cnt_loop(d2d, kr, acc):
        rounds = d2d.shape[0] // (_NT * kr)
        base = t * rounds * kr

        def body(g, carry):
            r0 = base + g * kr
            pltpu.sync_copy(d2d.at[pl.ds(r0, kr)], dbuf.at[pl.ds(0, kr)])
            ds_ = [pltpu.async_copy(ones, acc.at[dbuf.at[j]], sem, add=True)
                   for j in range(kr)]
            for dd in ds_:
                dd.wait()
            return carry

        lax.fori_loop(0, rounds, body, 0)

    bar = plsc.subcore_barrier
    zero1(acc_a, np_f)
    zero1(acc_b, nm_f)
    bar()

    @pl.when(c == 0)
    def _():
        cnt_loop(dpm, 8, acc_b)
        cnt_loop(dmp, 8, acc_a)

    @pl.when(c == 1)
    def _():
        cnt_loop(dpc, 2, acc_b)
        cnt_loop(dcp, 2, acc_a)

    bar()

    @pl.when(c == 0)
    def _():
        wb1(acc_b, nm_f, o_m)
        wb1(acc_a, np_f, o_p1)

    @pl.when(c == 1)
    def _():
        wb1(acc_b, nc_f, o_c)
        wb1(acc_a, np_f, o_p2)


def _sc_counts(dpm, dmp, dpc, dcp, zeros1, n_p, n_m, n_c):
    mesh = plsc.VectorSubcoreMesh(core_axis_name="c", subcore_axis_name="s")
    np_f, nm_f, nc_f = n_p + _trash(n_p), n_m + _trash(n_m), n_c + _trash(n_c)
    f = pl.kernel(
        functools.partial(_sc_counts_body, n_p, n_m, n_c),
        compiler_params=pltpu.CompilerParams(use_tc_tiling_on_sc=False),
        out_type=[jax.ShapeDtypeStruct((nm_f,), jnp.float32),
                  jax.ShapeDtypeStruct((np_f,), jnp.float32),
                  jax.ShapeDtypeStruct((nc_f,), jnp.float32),
                  jax.ShapeDtypeStruct((np_f,), jnp.float32)],
        mesh=mesh,
        scratch_types=[
            pltpu.VMEM_SHARED((np_f,), jnp.float32),
            pltpu.VMEM_SHARED((nm_f,), jnp.float32),
            pltpu.VMEM((8, 128), jnp.int32),
            pltpu.VMEM((128,), jnp.float32),
            pltpu.SemaphoreType.DMA,
            pltpu.VMEM(((n_p + _trash(n_p)) // _NT,), jnp.float32),
        ])
    return f(dpm, dmp, dpc, dcp, zeros1)


def _pad_edges(e, n_src, n_dst, e_pad):
    pad = e_pad - e.shape[1]
    idx = jnp.arange(pad, dtype=jnp.int32)
    ext = jnp.concatenate(
        [e, jnp.stack([idx % n_src, n_dst + (idx % _trash(n_dst))])], axis=1)
    return ext[0].reshape(-1, 128), ext[1].reshape(-1, 128)


def kernel(x_patient, x_med, x_com, edge_pm, edge_mp, edge_pc, edge_cp,
           proj_Wp, proj_bp, proj_Wm, proj_bm, proj_Wc, proj_bc,
           Wl, bl, Wr, cls_W, cls_b):
    n_p, n_m, n_c = x_patient.shape[0], x_med.shape[0], x_com.shape[0]
    EPM_PAD = _NT * 196 * _KR * 128
    EPC_PAD = _NT * 52 * _KR * 128

    spm, dpm = _pad_edges(edge_pm, n_p, n_m, EPM_PAD)
    smp, dmp = _pad_edges(edge_mp, n_m, n_p, EPM_PAD)
    spc, dpc = _pad_edges(edge_pc, n_p, n_c, EPC_PAD)
    scp, dcp = _pad_edges(edge_cp, n_c, n_p, EPC_PAD)
    eidx = (spm, dpm, smp, dmp, spc, dpc, scp, dcp)
    zeros2d = jnp.zeros((n_p + _trash(n_p), HH), jnp.float32)
    zeros1 = jnp.zeros((n_p + _trash(n_p),), jnp.float32)

    xp = _proj(x_patient, proj_Wp.T, proj_bp)
    xm = _proj(x_med, proj_Wm.T, proj_bm)
    xc = _proj(x_com, proj_Wc.T, proj_bc)

    cnt_m, cnt_p1, cnt_c, cnt_p2 = [
        a.reshape(-1, 1)
        for a in _sc_counts(dpm, dmp, dpc, dcp, zeros1, n_p, n_m, n_c)]

    s_pm, s_mp, s_pc, s_cp = _sc_agg(xp, xm, xc, eidx, zeros2d, n_p, n_m, n_c)
    xm1 = _comb1(s_pm, cnt_m, xm, Wl[0, 0].T, Wr[0, 0].T, bl[0, 0],
                 relu=True, n=n_m)
    xc1 = _comb1(s_pc, cnt_c, xc, Wl[0, 2].T, Wr[0, 2].T, bl[0, 2],
                 relu=True, n=n_c)
    xp1 = _comb2(s_mp, cnt_p1, s_cp, cnt_p2, xp, Wl[0, 1].T, Wl[0, 3].T,
                 (Wr[0, 1] + Wr[0, 3]).T, bl[0, 1] + bl[0, 3],
                 relu=True, n=n_p)

    s_pm2, s_mp2, s_pc2, s_cp2 = _sc_agg(xp1, xm1, xc1, eidx, zeros2d,
                                         n_p, n_m, n_c)

    cm_mean_m, cm_x_m = _colmean(s_pm2, cnt_m, xm1, n=n_m)
    cm_mean_c, cm_x_c = _colmean(s_pc2, cnt_c, xc1, n=n_c)

    w1, w2, w3 = cls_W[0, :H], cls_W[0, H:2 * H], cls_W[0, 2 * H:]
    a1 = Wl[1, 1].T @ w1
    a2 = Wl[1, 3].T @ w1
    av = (Wr[1, 1] + Wr[1, 3]).T @ w1
    g1, g2 = Wl[1, 0].T @ w2, Wr[1, 0].T @ w2
    h1, h2 = Wl[1, 2].T @ w3, Wr[1, 2].T @ w3
    s_scalar = (cm_mean_m[0] @ g1 + cm_x_m[0] @ g2
                + cm_mean_c[0] @ h1 + cm_x_c[0] @ h2
                + (bl[1, 1] + bl[1, 3]) @ w1 + bl[1, 0] @ w2 + bl[1, 2] @ w3
                + cls_b[0])
    amat = jnp.stack([a1, a2, av], axis=1)

    return _final(s_mp2, cnt_p1, s_cp2, cnt_p2, xp1, amat,
                  s_scalar.reshape(1, 1), n=n_p)


# speedup vs baseline: 15.4507x; 1.0791x over previous
import functools

import jax
import jax.numpy as jnp
from jax import lax
from jax.experimental import pallas as pl
from jax.experimental.pallas import tpu as pltpu, tpu_sc as plsc

H = 64
HH = 32
_PREC = lax.Precision.HIGHEST
_NT = 16


def _trash(n):
    t = (-n) % 128
    return t if t else 128


def _dot(a, b):
    return jnp.dot(a, b, preferred_element_type=jnp.float32, precision=_PREC)


def _proj_body(x_ref, w_ref, b_ref, o_ref):
    y = _dot(x_ref[...], w_ref[...]) + b_ref[...]
    o_ref[0] = y[:, :HH]
    o_ref[1] = y[:, HH:]


def _proj(x, wT, b, R=2000):
    n, d = x.shape
    return pl.pallas_call(
        _proj_body,
        grid=(n // R,),
        in_specs=[pl.BlockSpec((R, d), lambda i: (i, 0)),
                  pl.BlockSpec((d, H), lambda i: (0, 0)),
                  pl.BlockSpec((1, H), lambda i: (0, 0))],
        out_specs=pl.BlockSpec((2, R, HH), lambda i: (0, i, 0)),
        out_shape=jax.ShapeDtypeStruct((2, n + _trash(n), HH), jnp.float32),
    )(x, wT, b.reshape(1, H))


def _cat(ref):
    return jnp.concatenate([ref[0], ref[1]], axis=1)


def _mean(s_ref, cnt_ref):
    return _cat(s_ref) * (1.0 / jnp.maximum(cnt_ref[...], 1.0))


def _comb1_body(relu, s_ref, cnt_ref, x_ref, a_ref, bmat_ref, bias_ref, o_ref):
    y = _dot(_mean(s_ref, cnt_ref), a_ref[...]) + _dot(_cat(x_ref), bmat_ref[...]) + bias_ref[...]
    if relu:
        y = jnp.maximum(y, 0.0)
    o_ref[0] = y[:, :HH]
    o_ref[1] = y[:, HH:]


def _comb1(s, cnt, x, aT, bT, bias, relu, n, R=2000):
    return pl.pallas_call(
        functools.partial(_comb1_body, relu),
        grid=(n // R,),
        in_specs=[pl.BlockSpec((2, R, HH), lambda i: (0, i, 0)),
                  pl.BlockSpec((R, 1), lambda i: (i, 0)),
                  pl.BlockSpec((2, R, HH), lambda i: (0, i, 0)),
                  pl.BlockSpec((H, H), lambda i: (0, 0)),
                  pl.BlockSpec((H, H), lambda i: (0, 0)),
                  pl.BlockSpec((1, H), lambda i: (0, 0))],
        out_specs=pl.BlockSpec((2, R, HH), lambda i: (0, i, 0)),
        out_shape=jax.ShapeDtypeStruct((2, n + _trash(n), HH), jnp.float32),
    )(s, cnt, x, aT, bT, bias.reshape(1, H))


def _comb2_body(relu, s1_ref, c1_ref, s2_ref, c2_ref, x_ref, a1_ref, a2_ref,
                bmat_ref, bias_ref, o_ref):
    y = (_dot(_mean(s1_ref, c1_ref), a1_ref[...])
         + _dot(_mean(s2_ref, c2_ref), a2_ref[...])
         + _dot(_cat(x_ref), bmat_ref[...]) + bias_ref[...])
    if relu:
        y = jnp.maximum(y, 0.0)
    o_ref[0] = y[:, :HH]
    o_ref[1] = y[:, HH:]


def _comb2(s1, c1, s2, c2, x, a1T, a2T, bT, bias, relu, n, R=2000):
    return pl.pallas_call(
        functools.partial(_comb2_body, relu),
        grid=(n // R,),
        in_specs=[pl.BlockSpec((2, R, HH), lambda i: (0, i, 0)),
                  pl.BlockSpec((R, 1), lambda i: (i, 0)),
                  pl.BlockSpec((2, R, HH), lambda i: (0, i, 0)),
                  pl.BlockSpec((R, 1), lambda i: (i, 0)),
                  pl.BlockSpec((2, R, HH), lambda i: (0, i, 0)),
                  pl.BlockSpec((H, H), lambda i: (0, 0)),
                  pl.BlockSpec((H, H), lambda i: (0, 0)),
                  pl.BlockSpec((H, H), lambda i: (0, 0)),
                  pl.BlockSpec((1, H), lambda i: (0, 0))],
        out_specs=pl.BlockSpec((2, R, HH), lambda i: (0, i, 0)),
        out_shape=jax.ShapeDtypeStruct((2, n + _trash(n), HH), jnp.float32),
    )(s1, c1, s2, c2, x, a1T, a2T, bT, bias.reshape(1, H))


def _colmean_body(inv_n, s_ref, cnt_ref, x_ref, o1_ref, o2_ref):
    @pl.when(pl.program_id(0) == 0)
    def _():
        o1_ref[...] = jnp.zeros_like(o1_ref)
        o2_ref[...] = jnp.zeros_like(o2_ref)

    o1_ref[...] += jnp.sum(_mean(s_ref, cnt_ref), axis=0, keepdims=True) * inv_n
    o2_ref[...] += jnp.sum(_cat(x_ref), axis=0, keepdims=True) * inv_n


def _colmean(s, cnt, x, n, R=2000):
    return pl.pallas_call(
        functools.partial(_colmean_body, 1.0 / n),
        grid=(n // R,),
        in_specs=[pl.BlockSpec((2, R, HH), lambda i: (0, i, 0)),
                  pl.BlockSpec((R, 1), lambda i: (i, 0)),
                  pl.BlockSpec((2, R, HH), lambda i: (0, i, 0))],
        out_specs=[pl.BlockSpec((1, H), lambda i: (0, 0)),
                   pl.BlockSpec((1, H), lambda i: (0, 0))],
        out_shape=[jax.ShapeDtypeStruct((1, H), jnp.float32),
                   jax.ShapeDtypeStruct((1, H), jnp.float32)],
    )(s, cnt, x)


def _final_body(s1_ref, c1_ref, s2_ref, c2_ref, x_ref, amat_ref, const_ref, o_ref):
    y = (_dot(_mean(s1_ref, c1_ref), amat_ref[:, 0:1])
         + _dot(_mean(s2_ref, c2_ref), amat_ref[:, 1:2])
         + _dot(_cat(x_ref), amat_ref[:, 2:3]) + const_ref[...])
    o_ref[...] = y


def _final(s1, c1, s2, c2, x, amat, const, n, R=2000):
    return pl.pallas_call(
        _final_body,
        grid=(n // R,),
        in_specs=[pl.BlockSpec((2, R, HH), lambda i: (0, i, 0)),
                  pl.BlockSpec((R, 1), lambda i: (i, 0)),
                  pl.BlockSpec((2, R, HH), lambda i: (0, i, 0)),
                  pl.BlockSpec((R, 1), lambda i: (i, 0)),
                  pl.BlockSpec((2, R, HH), lambda i: (0, i, 0)),
                  pl.BlockSpec((H, 3), lambda i: (0, 0)),
                  pl.BlockSpec((1, 1), lambda i: (0, 0))],
        out_specs=pl.BlockSpec((R, 1), lambda i: (i, 0)),
        out_shape=jax.ShapeDtypeStruct((n, 1), jnp.float32),
    )(s1, c1, s2, c2, x, amat, const)


_KR = 2


def _sc_agg_body(n_p, n_m, n_c, xp_s, xm_s, xc_s, spm, dpm, smp, dmp, spc,
                 dpc, scp, dcp, zeros2d, o_pm, o_mp, o_pc, o_cp,
                 acc, gbuf, dbuf, rows, *sems16):
    semi = sems16[:8]
    semg = sems16[8:12]
    sems = sems16[12:16]
    c = lax.axis_index("c")
    t = lax.axis_index("s")
    np_f, nm_f, nc_f = n_p + _trash(n_p), n_m + _trash(n_m), n_c + _trash(n_c)

    def zero(n):
        k = n // _NT
        pltpu.sync_copy(zeros2d.at[pl.ds(t * k, k)], acc.at[pl.ds(t * k, k)])

    def writeback(n, out):
        k = n // _NT
        pltpu.sync_copy(acc.at[pl.ds(t * k, k)], out.at[c, pl.ds(t * k, k)])

    def edge_loop(s2d, d2d, tbl):
        rounds = s2d.shape[0] // _NT
        octs = rounds // 8
        base = t * rounds

        def fire_idx(g, sl):
            r0 = base + g
            pltpu.async_copy(s2d.at[r0], gbuf.at[sl], semi[sl])
            pltpu.async_copy(d2d.at[r0], dbuf.at[sl], semi[sl])

        def wait_idx(sl):
            pltpu.make_async_copy(s2d.at[0], gbuf.at[sl], semi[sl]).wait()
            pltpu.make_async_copy(d2d.at[0], dbuf.at[sl], semi[sl]).wait()

        def fire_g(isl, rsl):
            pltpu.async_copy(tbl.at[gbuf.at[isl]], rows.at[rsl], semg[rsl])

        def wait_g(rsl):
            pltpu.make_async_copy(tbl.at[gbuf.at[0]], rows.at[rsl], semg[rsl]).wait()

        def fire_s(isl, rsl):
            pltpu.async_copy(rows.at[rsl], acc.at[dbuf.at[isl]], sems[rsl], add=True)

        def wait_s(rsl):
            pltpu.make_async_copy(rows.at[rsl], acc.at[dbuf.at[0]], sems[rsl]).wait()

        for sl in range(4):
            fire_idx(sl, sl)

        def body(o, carry):
            g0 = 8 * o
            for k in range(4):
                wait_idx(k)

                @pl.when(o > 0)
                def _():
                    wait_s(k)

                fire_g(k, k)
            for k in range(4):
                fire_idx(g0 + 4 + k, 4 + k)
            for k in range(4):
                wait_g(k)
                fire_s(k, k)
            for k in range(4):
                wait_idx(4 + k)
                wait_s(k)
                fire_g(4 + k, k)

            @pl.when(o + 1 < octs)
            def _():
                for k in range(4):
                    fire_idx(g0 + 8 + k, k)

            for k in range(4):
                wait_g(k)
                fire_s(4 + k, k)
            return carry

        lax.fori_loop(0, octs, body, 0)
        for k in range(4):
            wait_s(k)

    bar = plsc.subcore_barrier
    for src2d, dst2d, x_hbm, n_out, out in (
            (spm, dpm, xp_s, nm_f, o_pm),
            (spc, dpc, xp_s, nc_f, o_pc),
            (smp, dmp, xm_s, np_f, o_mp),
            (scp, dcp, xc_s, np_f, o_cp)):
        zero(n_out)
        bar()
        edge_loop(src2d, dst2d, x_hbm.at[c])
        bar()
        writeback(n_out, out)
        bar()


def _sc_agg(xp_s, xm_s, xc_s, eidx, zeros2d, n_p, n_m, n_c):
    mesh = plsc.VectorSubcoreMesh(core_axis_name="c", subcore_axis_name="s")
    np_f, nm_f, nc_f = n_p + _trash(n_p), n_m + _trash(n_m), n_c + _trash(n_c)
    f = pl.kernel(
        functools.partial(_sc_agg_body, n_p, n_m, n_c),
        compiler_params=pltpu.CompilerParams(use_tc_tiling_on_sc=False),
        out_type=[jax.ShapeDtypeStruct((2, nm_f, HH), jnp.float32),
                  jax.ShapeDtypeStruct((2, np_f, HH), jnp.float32),
                  jax.ShapeDtypeStruct((2, nc_f, HH), jnp.float32),
                  jax.ShapeDtypeStruct((2, np_f, HH), jnp.float32)],
        mesh=mesh,
        scratch_types=[
            pltpu.VMEM_SHARED((np_f, HH), jnp.float32),
            pltpu.VMEM((8, 128), jnp.int32),
            pltpu.VMEM((8, 128), jnp.int32),
            pltpu.VMEM((4, 128, HH), jnp.float32),
        ] + [pltpu.SemaphoreType.DMA] * 16)
    return f(xp_s, xm_s, xc_s, *eidx, zeros2d)


def _sc_counts_body(n_p, n_m, n_c, dpm, dmp, dpc, dcp, zeros1,
                    o_m, o_p1, o_c, o_p2, acc_a, acc_b, dbuf, ones, sem, b1d):
    c = lax.axis_index("c")
    t = lax.axis_index("s")
    np_f, nm_f, nc_f = n_p + _trash(n_p), n_m + _trash(n_m), n_c + _trash(n_c)

    for i in range(8):
        ones[pl.ds(i * 16, 16)] = jnp.ones((16,), jnp.float32)

    def zero1(acc, n):
        k = n // _NT
        pltpu.sync_copy(zeros1.at[pl.ds(t * k, k)], b1d.at[pl.ds(0, k)])
        pltpu.sync_copy(b1d.at[pl.ds(0, k)], acc.at[pl.ds(t * k, k)])

    def wb1(acc, n, out):
        k = n // _NT
        pltpu.sync_copy(acc.at[pl.ds(t * k, k)], b1d.at[pl.ds(0, k)])
        pltpu.sync_copy(b1d.at[pl.ds(0, k)], out.at[pl.ds(t * k, k)])

    def cnt_loop(d2d, kr, acc):
        rounds = d2d.shape[0] // (_NT * kr)
        base = t * rounds * kr

        def body(g, carry):
            r0 = base + g * kr
            pltpu.sync_copy(d2d.at[pl.ds(r0, kr)], dbuf.at[pl.ds(0, kr)])
            ds_ = [pltpu.async_copy(ones, acc.at[dbuf.at[j]], sem, add=True)
                   for j in range(kr)]
            for dd in ds_:
                dd.wait()
            return carry

        lax.fori_loop(0, rounds, body, 0)

    bar = plsc.subcore_barrier
    zero1(acc_a, np_f)
    zero1(acc_b, nm_f)
    bar()

    @pl.when(c == 0)
    def _():
        cnt_loop(dpm, 8, acc_b)
        cnt_loop(dmp, 8, acc_a)

    @pl.when(c == 1)
    def _():
        cnt_loop(dpc, 2, acc_b)
        cnt_loop(dcp, 2, acc_a)

    bar()

    @pl.when(c == 0)
    def _():
        wb1(acc_b, nm_f, o_m)
        wb1(acc_a, np_f, o_p1)

    @pl.when(c == 1)
    def _():
        wb1(acc_b, nc_f, o_c)
        wb1(acc_a, np_f, o_p2)


def _sc_counts(dpm, dmp, dpc, dcp, zeros1, n_p, n_m, n_c):
    mesh = plsc.VectorSubcoreMesh(core_axis_name="c", subcore_axis_name="s")
    np_f, nm_f, nc_f = n_p + _trash(n_p), n_m + _trash(n_m), n_c + _trash(n_c)
    f = pl.kernel(
        functools.partial(_sc_counts_body, n_p, n_m, n_c),
        compiler_params=pltpu.CompilerParams(use_tc_tiling_on_sc=False),
        out_type=[jax.ShapeDtypeStruct((nm_f,), jnp.float32),
                  jax.ShapeDtypeStruct((np_f,), jnp.float32),
                  jax.ShapeDtypeStruct((nc_f,), jnp.float32),
                  jax.ShapeDtypeStruct((np_f,), jnp.float32)],
        mesh=mesh,
        scratch_types=[
            pltpu.VMEM_SHARED((np_f,), jnp.float32),
            pltpu.VMEM_SHARED((nm_f,), jnp.float32),
            pltpu.VMEM((8, 128), jnp.int32),
            pltpu.VMEM((128,), jnp.float32),
            pltpu.SemaphoreType.DMA,
            pltpu.VMEM(((n_p + _trash(n_p)) // _NT,), jnp.float32),
        ])
    return f(dpm, dmp, dpc, dcp, zeros1)


def _pad_edges(e, n_src, n_dst, e_pad):
    pad = e_pad - e.shape[1]
    idx = jnp.arange(pad, dtype=jnp.int32)
    ext = jnp.concatenate(
        [e, jnp.stack([idx % n_src, n_dst + (idx % _trash(n_dst))])], axis=1)
    return ext[0].reshape(-1, 128), ext[1].reshape(-1, 128)


def kernel(x_patient, x_med, x_com, edge_pm, edge_mp, edge_pc, edge_cp,
           proj_Wp, proj_bp, proj_Wm, proj_bm, proj_Wc, proj_bc,
           Wl, bl, Wr, cls_W, cls_b):
    n_p, n_m, n_c = x_patient.shape[0], x_med.shape[0], x_com.shape[0]
    EPM_PAD = _NT * 196 * _KR * 128
    EPC_PAD = _NT * 52 * _KR * 128

    spm, dpm = _pad_edges(edge_pm, n_p, n_m, EPM_PAD)
    smp, dmp = _pad_edges(edge_mp, n_m, n_p, EPM_PAD)
    spc, dpc = _pad_edges(edge_pc, n_p, n_c, EPC_PAD)
    scp, dcp = _pad_edges(edge_cp, n_c, n_p, EPC_PAD)
    eidx = (spm, dpm, smp, dmp, spc, dpc, scp, dcp)
    zeros2d = jnp.zeros((n_p + _trash(n_p), HH), jnp.float32)
    zeros1 = jnp.zeros((n_p + _trash(n_p),), jnp.float32)

    xp = _proj(x_patient, proj_Wp.T, proj_bp)
    xm = _proj(x_med, proj_Wm.T, proj_bm)
    xc = _proj(x_com, proj_Wc.T, proj_bc)

    cnt_m, cnt_p1, cnt_c, cnt_p2 = [
        a.reshape(-1, 1)
        for a in _sc_counts(dpm, dmp, dpc, dcp, zeros1, n_p, n_m, n_c)]

    s_pm, s_mp, s_pc, s_cp = _sc_agg(xp, xm, xc, eidx, zeros2d, n_p, n_m, n_c)
    xm1 = _comb1(s_pm, cnt_m, xm, Wl[0, 0].T, Wr[0, 0].T, bl[0, 0],
                 relu=True, n=n_m)
    xc1 = _comb1(s_pc, cnt_c, xc, Wl[0, 2].T, Wr[0, 2].T, bl[0, 2],
                 relu=True, n=n_c)
    xp1 = _comb2(s_mp, cnt_p1, s_cp, cnt_p2, xp, Wl[0, 1].T, Wl[0, 3].T,
                 (Wr[0, 1] + Wr[0, 3]).T, bl[0, 1] + bl[0, 3],
                 relu=True, n=n_p)

    s_pm2, s_mp2, s_pc2, s_cp2 = _sc_agg(xp1, xm1, xc1, eidx, zeros2d,
                                         n_p, n_m, n_c)

    cm_mean_m, cm_x_m = _colmean(s_pm2, cnt_m, xm1, n=n_m)
    cm_mean_c, cm_x_c = _colmean(s_pc2, cnt_c, xc1, n=n_c)

    w1, w2, w3 = cls_W[0, :H], cls_W[0, H:2 * H], cls_W[0, 2 * H:]
    a1 = Wl[1, 1].T @ w1
    a2 = Wl[1, 3].T @ w1
    av = (Wr[1, 1] + Wr[1, 3]).T @ w1
    g1, g2 = Wl[1, 0].T @ w2, Wr[1, 0].T @ w2
    h1, h2 = Wl[1, 2].T @ w3, Wr[1, 2].T @ w3
    s_scalar = (cm_mean_m[0] @ g1 + cm_x_m[0] @ g2
                + cm_mean_c[0] @ h1 + cm_x_c[0] @ h2
                + (bl[1, 1] + bl[1, 3]) @ w1 + bl[1, 0] @ w2 + bl[1, 2] @ w3
                + cls_b[0])
    amat = jnp.stack([a1, a2, av], axis=1)

    return _final(s_mp2, cnt_p1, s_cp2, cnt_p2, xp1, amat,
                  s_scalar.reshape(1, 1), n=n_p)
